# Initial kernel scaffold; baseline (speedup 1.0000x reference)
import jax
import jax.numpy as jnp
from jax.experimental import pallas as pl


def kernel(x, edge_index, edge_attr, batch, nn0_w1, nn0_b1, nn0_w2, nn0_b2, root0, bias0, nn1_w1, nn1_b1, nn1_w2, nn1_b2, root1, bias1, fc0_w, fc0_b, fc1_w, fc1_b, fc2_w, fc2_b):
    raise NotImplementedError("write your pallas kernel here")


# speedup vs baseline: 1.4458x; 1.4458x over previous
import functools

import jax
import jax.numpy as jnp
from jax import lax
from jax.experimental import pallas as pl
from jax.experimental.pallas import tpu as pltpu
from jax.experimental.pallas import tpu_sc as plsc

N_NODES = 10000
N_EDGES = 160000
F_IN = 128
B_IN = 16
DIM = 16
N_GRAPHS = 64
LANE = 128

NUM_CORES = 2
NUM_SUBCORES = 16
NW = NUM_CORES * NUM_SUBCORES
CHUNK = 128
CHUNKS_PER_W = 40
E_PER_W = CHUNK * CHUNKS_PER_W
E_PAD = NW * E_PER_W
N_ACC = 10240

BLK = 1024
N_EDGE_BLKS = E_PAD // BLK


def _make_sc_gather():
  mesh = plsc.VectorSubcoreMesh(core_axis_name="c", subcore_axis_name="s")

  @functools.partial(
      pl.kernel,
      mesh=mesh,
      out_type=jax.ShapeDtypeStruct((E_PAD, LANE), jnp.float32),
      scratch_types=[
          pltpu.VMEM((CHUNKS_PER_W, CHUNK), jnp.int32),
          pltpu.VMEM((2, CHUNK, LANE), jnp.float32),
          pltpu.SemaphoreType.DMA,
          pltpu.SemaphoreType.DMA,
      ],
  )
  def gather_kernel(idx_hbm, table_hbm, out_hbm, idx_v, buf, gsem, wsem):
    c = lax.axis_index("c")
    s = lax.axis_index("s")
    wid = s * NUM_CORES + c
    row0 = wid * CHUNKS_PER_W
    ebase = wid * E_PER_W
    pltpu.sync_copy(idx_hbm.at[pl.ds(row0, CHUNKS_PER_W)], idx_v)
    for j in range(CHUNKS_PER_W):
      pltpu.async_copy(table_hbm.at[idx_v.at[j]], buf.at[j % 2], gsem).wait()
      pltpu.async_copy(buf.at[j % 2],
                       out_hbm.at[pl.ds(ebase + j * CHUNK, CHUNK)], wsem).wait()

  return gather_kernel


HALF = N_ACC // 2
ACC_W = HALF * DIM
RED = ACC_W // NUM_SUBCORES


def _make_sc_scatter_add():
  mesh = plsc.VectorSubcoreMesh(core_axis_name="c", subcore_axis_name="s")

  @functools.partial(
      pl.kernel,
      mesh=mesh,
      out_type=jax.ShapeDtypeStruct((NW, 2 * ACC_W), jnp.float32),
      compiler_params=pltpu.CompilerParams(needs_layout_passes=False),
      scratch_types=[
          pltpu.VMEM((CHUNKS_PER_W, CHUNK), jnp.int32),
          pltpu.VMEM((CHUNK, DIM), jnp.float32),
          pltpu.VMEM((CHUNK, DIM), jnp.float32),
          pltpu.VMEM((ACC_W,), jnp.float32),
          pltpu.SemaphoreType.DMA,
          pltpu.SemaphoreType.DMA,
      ],
  )
  def scatter_kernel(idx_hbm, msg_hbm, out_hbm, idx_v, m_a, m_b, acc,
                     sem_a, sem_b):
    c = lax.axis_index("c")
    s = lax.axis_index("s")
    wid = s * NUM_CORES + c
    ebase = wid * E_PER_W
    pltpu.sync_copy(idx_hbm.at[pl.ds(wid * CHUNKS_PER_W, CHUNKS_PER_W)], idx_v)
    lanes = lax.iota(jnp.int32, 16)
    zvec = jnp.zeros((16,), jnp.float32)

    def process_chunk(j, mb, lo):
      @pl.loop(0, CHUNK // 16)
      def _grp(g):
        dvec = idx_v[j, pl.ds(g * 16, 16)]
        for l in range(16):
          dd = dvec[l] - lo
          ddv = jnp.full((16,), dd, jnp.int32)
          mask = (ddv >= 0) & (ddv < HALF)
          plsc.addupdate_scatter(acc, [ddv * DIM + lanes], mb[g * 16 + l, :],
                                 mask=mask)

    def start_load(j, buf, sem):
      return pltpu.async_copy(msg_hbm.at[pl.ds(ebase + j * CHUNK, CHUNK)],
                              buf, sem)

    for p in range(2):
      lo = p * HALF

      @pl.loop(0, ACC_W // 16, unroll=8)
      def _zero(k):
        acc[pl.ds(k * 16, 16)] = zvec

      start_load(0, m_a, sem_a)

      @pl.loop(0, CHUNKS_PER_W // 2)
      def _pair(t):
        j = t * 2
        start_load(j + 1, m_b, sem_b)
        pltpu.make_async_copy(msg_hbm.at[pl.ds(ebase, CHUNK)], m_a,
                              sem_a).wait()
        process_chunk(j, m_a, lo)

        @pl.when(t + 1 < CHUNKS_PER_W // 2)
        def _():
          start_load(j + 2, m_a, sem_a)

        pltpu.make_async_copy(msg_hbm.at[pl.ds(ebase, CHUNK)], m_b,
                              sem_b).wait()
        process_chunk(j + 1, m_b, lo)

      pltpu.sync_copy(acc, out_hbm.at[wid, pl.ds(p * ACC_W, ACC_W)])

  return scatter_kernel


def _reduce_body(parts_ref, out_ref):
  out_ref[...] = jnp.sum(parts_ref[...], axis=0, keepdims=True)


def _reduce_call(parts):
  blk = 8192
  return pl.pallas_call(
      _reduce_body,
      grid=(2 * ACC_W // blk,),
      in_specs=[pl.BlockSpec((NW, blk), lambda i: (0, i))],
      out_specs=pl.BlockSpec((1, blk), lambda i: (0, i)),
      out_shape=jax.ShapeDtypeStruct((1, 2 * ACC_W), jnp.float32),
  )(parts)


def _edge_valid(blk):
  rows = lax.broadcasted_iota(jnp.int32, (blk, 1), 0) + pl.program_id(0) * blk
  return (rows < N_EDGES).astype(jnp.float32)


def _msg0_body(xg_ref, ea_ref, w1_ref, b1_ref, w2cat_ref, b2m_ref, out_ref):
  xg = xg_ref[...]
  ea = ea_ref[...]
  h = jnp.maximum(
      lax.dot_general(ea, w1_ref[...], (((1,), (1,)), ((), ()))) + b1_ref[...],
      0.0)
  cols = []
  for o in range(DIM // 2):
    p = jnp.dot(xg, w2cat_ref[:, o * 128:(o + 1) * 128],
                preferred_element_type=jnp.float32)
    cols.append(jnp.sum(p * h, axis=1, keepdims=True))
  zeros = jnp.zeros((xg.shape[0], DIM // 2), jnp.float32)
  msg = jnp.concatenate(cols + [zeros], axis=1) + jnp.dot(
      xg, b2m_ref[...], preferred_element_type=jnp.float32)
  out_ref[...] = msg * _edge_valid(xg.shape[0])


def _msg1_body(x1g_ref, ea_ref, w1_ref, b1_ref, w2_ref, b2_ref, out_ref):
  ea = ea_ref[...]
  h = jnp.maximum(
      lax.dot_general(ea, w1_ref[...], (((1,), (1,)), ((), ()))) + b1_ref[...],
      0.0)
  w1e = lax.dot_general(h, w2_ref[...], (((1,), (1,)), ((), ())))
  msg = jnp.zeros((ea.shape[0], DIM), jnp.float32)
  for i in range(DIM // 2):
    msg = msg + x1g_ref[:, i:i + 1] * (
        w1e[:, i * DIM:(i + 1) * DIM] + b2_ref[0:1, i * DIM:(i + 1) * DIM])
  out_ref[...] = msg * _edge_valid(ea.shape[0])


def _node0_body(x_ref, aggp_ref, root_ref, bias_ref, out_ref):
  agg = aggp_ref[:N_NODES, :]
  x1 = jnp.maximum(
      jnp.dot(x_ref[...], root_ref[...], preferred_element_type=jnp.float32)
      + agg + bias_ref[...], 0.0)
  out_ref[...] = jnp.concatenate(
      [x1, jnp.zeros((N_NODES, LANE - DIM), jnp.float32)], axis=1)


def _readout_body(x1p_ref, aggp_ref, root_ref, bias_ref, batch_ref,
                  fc0w_ref, fc0b_ref, fc1w_ref, fc1b_ref, fc2w_ref, fc2b_ref,
                  out_ref):
  agg = aggp_ref[:N_NODES, :]
  x2 = jnp.maximum(
      jnp.dot(x1p_ref[...], root_ref[...], preferred_element_type=jnp.float32)
      + agg + bias_ref[...], 0.0)
  gids = lax.broadcasted_iota(jnp.int32, (N_GRAPHS, N_NODES), 0)
  onehot = (gids == batch_ref[...]).astype(jnp.float32)
  x2a = jnp.concatenate([x2, jnp.ones((N_NODES, 1), jnp.float32)], axis=1)
  seg = jnp.dot(onehot, x2a, preferred_element_type=jnp.float32)
  cnt = seg[:, DIM:DIM + 1]
  g = seg[:, :DIM] / jnp.maximum(cnt, 1.0)
  g = jnp.maximum(
      lax.dot_general(g, fc0w_ref[...], (((1,), (1,)), ((), ())))
      + fc0b_ref[...], 0.0)
  g = jnp.maximum(
      lax.dot_general(g, fc1w_ref[...], (((1,), (1,)), ((), ())))
      + fc1b_ref[...], 0.0)
  out_ref[...] = jnp.sum(g * fc2w_ref[...], axis=1, keepdims=True) + fc2b_ref[0, 0]


def _full_spec(shape):
  nd = len(shape)
  return pl.BlockSpec(shape, lambda i=0, *, _n=nd: (0,) * _n)


def _msg0_call(xg, ea, w1, b1, w2cat, b2m):
  in_specs = [pl.BlockSpec((BLK, LANE), lambda i: (i, 0)),
              pl.BlockSpec((BLK, B_IN), lambda i: (i, 0)),
              _full_spec(w1.shape), _full_spec(b1.shape),
              _full_spec(w2cat.shape), _full_spec(b2m.shape)]
  return pl.pallas_call(
      _msg0_body,
      grid=(N_EDGE_BLKS,),
      in_specs=in_specs,
      out_specs=pl.BlockSpec((BLK, DIM), lambda i: (i, 0)),
      out_shape=jax.ShapeDtypeStruct((E_PAD, DIM), jnp.float32),
  )(xg, ea, w1, b1, w2cat, b2m)


def _msg1_call(x1g, ea, w1, b1, w2, b2):
  in_specs = [pl.BlockSpec((BLK, LANE), lambda i: (i, 0)),
              pl.BlockSpec((BLK, B_IN), lambda i: (i, 0)),
              _full_spec(w1.shape), _full_spec(b1.shape),
              _full_spec(w2.shape), _full_spec(b2.shape)]
  return pl.pallas_call(
      _msg1_body,
      grid=(N_EDGE_BLKS,),
      in_specs=in_specs,
      out_specs=pl.BlockSpec((BLK, DIM), lambda i: (i, 0)),
      out_shape=jax.ShapeDtypeStruct((E_PAD, DIM), jnp.float32),
  )(x1g, ea, w1, b1, w2, b2)


def _node0_call(x, aggp, rootp, biasp):
  return pl.pallas_call(
      _node0_body,
      in_specs=[_full_spec(x.shape), _full_spec(aggp.shape),
                _full_spec(rootp.shape), _full_spec(biasp.shape)],
      out_specs=_full_spec((N_NODES, LANE)),
      out_shape=jax.ShapeDtypeStruct((N_NODES, LANE), jnp.float32),
  )(x, aggp, rootp, biasp)


def _readout_call(x1p, aggp, rootp, biasp, batch_row, fc0w, fc0b, fc1w, fc1b,
                  fc2w, fc2b):
  args = (x1p, aggp, rootp, biasp, batch_row, fc0w, fc0b, fc1w, fc1b, fc2w,
          fc2b)
  return pl.pallas_call(
      _readout_body,
      in_specs=[_full_spec(a.shape) for a in args],
      out_specs=_full_spec((N_GRAPHS, 1)),
      out_shape=jax.ShapeDtypeStruct((N_GRAPHS, 1), jnp.float32),
  )(*args)


_make_sc_gather = functools.lru_cache(maxsize=None)(_make_sc_gather)
_make_sc_scatter_add = functools.lru_cache(maxsize=None)(_make_sc_scatter_add)


@jax.jit
def kernel(x, edge_index, edge_attr, batch, nn0_w1, nn0_b1, nn0_w2, nn0_b2,
           root0, bias0, nn1_w1, nn1_b1, nn1_w2, nn1_b2, root1, bias1,
           fc0_w, fc0_b, fc1_w, fc1_b, fc2_w, fc2_b):
  pad_e = E_PAD - N_EDGES
  src = jnp.concatenate(
      [edge_index[0], jnp.zeros((pad_e,), jnp.int32)]).reshape(-1, CHUNK)
  dst = jnp.concatenate(
      [edge_index[1], jnp.full((pad_e,), N_NODES, jnp.int32)]).reshape(-1, CHUNK)
  ea = jnp.concatenate(
      [edge_attr, jnp.zeros((pad_e, B_IN), jnp.float32)], axis=0)
  w2cat0 = nn0_w2.reshape(F_IN, (DIM // 2) * F_IN)
  b2m0 = jnp.pad(nn0_b2.reshape(F_IN, DIM // 2), ((0, 0), (0, DIM // 2)))
  root0p = jnp.pad(root0, ((0, 0), (0, DIM // 2)))
  bias0p = jnp.pad(bias0, (0, DIM // 2)).reshape(1, DIM)
  root1p = jnp.pad(root1, ((0, LANE - DIM // 2), (0, 0)))
  bias1p = bias1.reshape(1, DIM)
  b1r0 = nn0_b1.reshape(1, 128)
  b1r1 = nn1_b1.reshape(1, 128)
  b2r1 = nn1_b2.reshape(1, 128)
  batch_row = batch.reshape(1, N_NODES)

  sc_gather = _make_sc_gather()
  sc_scatter = _make_sc_scatter_add()

  xg = sc_gather(src, x)
  msg0 = _msg0_call(xg, ea, nn0_w1, b1r0, w2cat0, b2m0)
  agg0 = _reduce_call(sc_scatter(dst, msg0)).reshape(N_ACC, DIM)
  x1p = _node0_call(x, agg0, root0p, bias0p)

  x1g = sc_gather(src, x1p)
  msg1 = _msg1_call(x1g, ea, nn1_w1, b1r1, nn1_w2, b2r1)
  agg1 = _reduce_call(sc_scatter(dst, msg1)).reshape(N_ACC, DIM)

  out = _readout_call(x1p, agg1, root1p, bias1p, batch_row,
                      fc0_w, fc0_b.reshape(1, -1),
                      fc1_w, fc1_b.reshape(1, -1),
                      fc2_w, fc2_b.reshape(1, -1))
  return out.reshape(-1)


# speedup vs baseline: 1.6722x; 1.1566x over previous
import functools

import jax
import jax.numpy as jnp
from jax import lax
from jax.experimental import pallas as pl
from jax.experimental.pallas import tpu as pltpu
from jax.experimental.pallas import tpu_sc as plsc

N_NODES = 10000
N_EDGES = 160000
F_IN = 128
B_IN = 16
DIM = 16
N_GRAPHS = 64
LANE = 128

NUM_CORES = 2
NUM_SUBCORES = 16
NW = NUM_CORES * NUM_SUBCORES
CHUNK = 128
CHUNKS_PER_W = 40
E_PER_W = CHUNK * CHUNKS_PER_W
E_PAD = NW * E_PER_W
N_ACC = 10240

BLK = 1024
N_EDGE_BLKS = E_PAD // BLK


def _make_sc_gather():
  mesh = plsc.VectorSubcoreMesh(core_axis_name="c", subcore_axis_name="s")

  @functools.partial(
      pl.kernel,
      mesh=mesh,
      out_type=jax.ShapeDtypeStruct((E_PAD, LANE), jnp.float32),
      scratch_types=[
          pltpu.VMEM((CHUNKS_PER_W, CHUNK), jnp.int32),
          pltpu.VMEM((4, CHUNK, LANE), jnp.float32),
          pltpu.SemaphoreType.DMA,
          pltpu.SemaphoreType.DMA,
          pltpu.SemaphoreType.DMA,
          pltpu.SemaphoreType.DMA,
          pltpu.SemaphoreType.DMA,
          pltpu.SemaphoreType.DMA,
          pltpu.SemaphoreType.DMA,
          pltpu.SemaphoreType.DMA,
      ],
  )
  def gather_kernel(idx_hbm, table_hbm, out_hbm, idx_v, buf, g0, g1, g2, g3,
                    w0, w1, w2, w3):
    c = lax.axis_index("c")
    s = lax.axis_index("s")
    wid = s * NUM_CORES + c
    row0 = wid * CHUNKS_PER_W
    ebase = wid * E_PER_W
    gsems = (g0, g1, g2, g3)
    wsems = (w0, w1, w2, w3)
    pltpu.sync_copy(idx_hbm.at[pl.ds(row0, CHUNKS_PER_W)], idx_v)
    gd, wd = {}, {}
    for j in range(CHUNKS_PER_W):
      b = j % 4
      if j >= 4:
        wd[j - 4].wait()
      gd[j] = pltpu.async_copy(table_hbm.at[idx_v.at[j]], buf.at[b], gsems[b])
      if j >= 1:
        gd[j - 1].wait()
        wd[j - 1] = pltpu.async_copy(
            buf.at[(j - 1) % 4],
            out_hbm.at[pl.ds(ebase + (j - 1) * CHUNK, CHUNK)],
            wsems[(j - 1) % 4])
    last = CHUNKS_PER_W - 1
    gd[last].wait()
    wd[last] = pltpu.async_copy(
        buf.at[last % 4], out_hbm.at[pl.ds(ebase + last * CHUNK, CHUNK)],
        wsems[last % 4])
    for j in range(CHUNKS_PER_W - 4, CHUNKS_PER_W):
      wd[j].wait()

  return gather_kernel


HALF = N_ACC // 2
ACC_W = HALF * DIM
RED = ACC_W // NUM_SUBCORES


def _make_sc_scatter_add(width):
  mesh = plsc.VectorSubcoreMesh(core_axis_name="c", subcore_axis_name="s")
  n_pass = 1 if width == 8 else 2
  rows = N_ACC if width == 8 else HALF

  @functools.partial(
      pl.kernel,
      mesh=mesh,
      out_type=jax.ShapeDtypeStruct((NW, n_pass * ACC_W), jnp.float32),
      compiler_params=pltpu.CompilerParams(needs_layout_passes=False),
      scratch_types=[
          pltpu.VMEM((CHUNKS_PER_W, CHUNK), jnp.int32),
          pltpu.VMEM((CHUNK, DIM), jnp.float32),
          pltpu.VMEM((CHUNK, DIM), jnp.float32),
          pltpu.VMEM((ACC_W,), jnp.float32),
          pltpu.SemaphoreType.DMA,
          pltpu.SemaphoreType.DMA,
      ],
  )
  def scatter_kernel(idx_hbm, msg_hbm, out_hbm, idx_v, m_a, m_b, acc,
                     sem_a, sem_b):
    c = lax.axis_index("c")
    s = lax.axis_index("s")
    wid = s * NUM_CORES + c
    ebase = wid * E_PER_W
    pltpu.sync_copy(idx_hbm.at[pl.ds(wid * CHUNKS_PER_W, CHUNKS_PER_W)], idx_v)
    lanes = lax.iota(jnp.int32, 16)
    zvec = jnp.zeros((16,), jnp.float32)
    const_mask = lanes < width

    def process_chunk(j, mb, lo):
      @pl.loop(0, CHUNK // 16)
      def _grp(g):
        dvec = idx_v[j, pl.ds(g * 16, 16)]
        for l in range(16):
          dd = dvec[l] - lo
          ddv = jnp.full((16,), dd, jnp.int32)
          if n_pass == 1:
            mask = const_mask
          else:
            mask = (ddv >= 0) & (ddv < rows)
          plsc.addupdate_scatter(acc, [ddv * width + lanes],
                                 mb[g * 16 + l, :], mask=mask)

    def start_load(j, buf, sem):
      return pltpu.async_copy(msg_hbm.at[pl.ds(ebase + j * CHUNK, CHUNK)],
                              buf, sem)

    for p in range(n_pass):
      lo = p * rows

      @pl.loop(0, ACC_W // 16, unroll=8)
      def _zero(k):
        acc[pl.ds(k * 16, 16)] = zvec

      start_load(0, m_a, sem_a)

      @pl.loop(0, CHUNKS_PER_W // 2)
      def _pair(t):
        j = t * 2
        start_load(j + 1, m_b, sem_b)
        pltpu.make_async_copy(msg_hbm.at[pl.ds(ebase, CHUNK)], m_a,
                              sem_a).wait()
        process_chunk(j, m_a, lo)

        @pl.when(t + 1 < CHUNKS_PER_W // 2)
        def _():
          start_load(j + 2, m_a, sem_a)

        pltpu.make_async_copy(msg_hbm.at[pl.ds(ebase, CHUNK)], m_b,
                              sem_b).wait()
        process_chunk(j + 1, m_b, lo)

      pltpu.sync_copy(acc, out_hbm.at[wid, pl.ds(p * ACC_W, ACC_W)])

  return scatter_kernel


def _reduce_body(parts_ref, out_ref):
  out_ref[...] = jnp.sum(parts_ref[...], axis=0, keepdims=True)


def _reduce_call(parts):
  blk = 8192
  total = parts.shape[1]
  return pl.pallas_call(
      _reduce_body,
      grid=(total // blk,),
      in_specs=[pl.BlockSpec((NW, blk), lambda i: (0, i))],
      out_specs=pl.BlockSpec((1, blk), lambda i: (0, i)),
      out_shape=jax.ShapeDtypeStruct((1, total), jnp.float32),
  )(parts)


def _edge_valid(blk):
  rows = lax.broadcasted_iota(jnp.int32, (blk, 1), 0) + pl.program_id(0) * blk
  return (rows < N_EDGES).astype(jnp.float32)


def _msg0_body(xg_ref, ea_ref, w1_ref, b1_ref, w2cat_ref, b2m_ref, out_ref):
  xg = xg_ref[...]
  ea = ea_ref[...]
  h = jnp.maximum(
      lax.dot_general(ea, w1_ref[...], (((1,), (1,)), ((), ()))) + b1_ref[...],
      0.0)
  cols = []
  for o in range(DIM // 2):
    p = jnp.dot(xg, w2cat_ref[:, o * 128:(o + 1) * 128],
                preferred_element_type=jnp.float32)
    cols.append(jnp.sum(p * h, axis=1, keepdims=True))
  zeros = jnp.zeros((xg.shape[0], DIM // 2), jnp.float32)
  msg = jnp.concatenate(cols + [zeros], axis=1) + jnp.dot(
      xg, b2m_ref[...], preferred_element_type=jnp.float32)
  out_ref[...] = msg * _edge_valid(xg.shape[0])


def _msg1_body(x1g_ref, ea_ref, w1_ref, b1_ref, w2_ref, b2_ref, out_ref):
  ea = ea_ref[...]
  h = jnp.maximum(
      lax.dot_general(ea, w1_ref[...], (((1,), (1,)), ((), ()))) + b1_ref[...],
      0.0)
  w1e = lax.dot_general(h, w2_ref[...], (((1,), (1,)), ((), ())))
  msg = jnp.zeros((ea.shape[0], DIM), jnp.float32)
  for i in range(DIM // 2):
    msg = msg + x1g_ref[:, i:i + 1] * (
        w1e[:, i * DIM:(i + 1) * DIM] + b2_ref[0:1, i * DIM:(i + 1) * DIM])
  out_ref[...] = msg * _edge_valid(ea.shape[0])


def _node0_body(x_ref, aggp_ref, root_ref, bias_ref, out_ref):
  agg = aggp_ref[:N_NODES, :]
  x1 = jnp.maximum(
      jnp.dot(x_ref[...], root_ref[...], preferred_element_type=jnp.float32)
      + agg + bias_ref[...], 0.0)
  out_ref[...] = jnp.concatenate(
      [x1, jnp.zeros((N_NODES, LANE - DIM // 2), jnp.float32)], axis=1)


def _readout_body(x1p_ref, aggp_ref, root_ref, bias_ref, batch_ref,
                  fc0w_ref, fc0b_ref, fc1w_ref, fc1b_ref, fc2w_ref, fc2b_ref,
                  out_ref):
  agg = aggp_ref[:N_NODES, :]
  x2 = jnp.maximum(
      jnp.dot(x1p_ref[...], root_ref[...], preferred_element_type=jnp.float32)
      + agg + bias_ref[...], 0.0)
  gids = lax.broadcasted_iota(jnp.int32, (N_GRAPHS, N_NODES), 0)
  onehot = (gids == batch_ref[...]).astype(jnp.float32)
  x2a = jnp.concatenate([x2, jnp.ones((N_NODES, 1), jnp.float32)], axis=1)
  seg = jnp.dot(onehot, x2a, preferred_element_type=jnp.float32)
  cnt = seg[:, DIM:DIM + 1]
  g = seg[:, :DIM] / jnp.maximum(cnt, 1.0)
  g = jnp.maximum(
      lax.dot_general(g, fc0w_ref[...], (((1,), (1,)), ((), ())))
      + fc0b_ref[...], 0.0)
  g = jnp.maximum(
      lax.dot_general(g, fc1w_ref[...], (((1,), (1,)), ((), ())))
      + fc1b_ref[...], 0.0)
  out_ref[...] = jnp.sum(g * fc2w_ref[...], axis=1, keepdims=True) + fc2b_ref[0, 0]


def _full_spec(shape):
  nd = len(shape)
  return pl.BlockSpec(shape, lambda i=0, *, _n=nd: (0,) * _n)


def _msg0_call(xg, ea, w1, b1, w2cat, b2m):
  in_specs = [pl.BlockSpec((BLK, LANE), lambda i: (i, 0)),
              pl.BlockSpec((BLK, B_IN), lambda i: (i, 0)),
              _full_spec(w1.shape), _full_spec(b1.shape),
              _full_spec(w2cat.shape), _full_spec(b2m.shape)]
  return pl.pallas_call(
      _msg0_body,
      grid=(N_EDGE_BLKS,),
      in_specs=in_specs,
      out_specs=pl.BlockSpec((BLK, DIM), lambda i: (i, 0)),
      out_shape=jax.ShapeDtypeStruct((E_PAD, DIM), jnp.float32),
  )(xg, ea, w1, b1, w2cat, b2m)


def _msg1_call(x1g, ea, w1, b1, w2, b2):
  in_specs = [pl.BlockSpec((BLK, LANE), lambda i: (i, 0)),
              pl.BlockSpec((BLK, B_IN), lambda i: (i, 0)),
              _full_spec(w1.shape), _full_spec(b1.shape),
              _full_spec(w2.shape), _full_spec(b2.shape)]
  return pl.pallas_call(
      _msg1_body,
      grid=(N_EDGE_BLKS,),
      in_specs=in_specs,
      out_specs=pl.BlockSpec((BLK, DIM), lambda i: (i, 0)),
      out_shape=jax.ShapeDtypeStruct((E_PAD, DIM), jnp.float32),
  )(x1g, ea, w1, b1, w2, b2)


def _node0_call(x, aggp, rootp, biasp):
  return pl.pallas_call(
      _node0_body,
      in_specs=[_full_spec(x.shape), _full_spec(aggp.shape),
                _full_spec(rootp.shape), _full_spec(biasp.shape)],
      out_specs=_full_spec((N_NODES, LANE)),
      out_shape=jax.ShapeDtypeStruct((N_NODES, LANE), jnp.float32),
  )(x, aggp, rootp, biasp)


def _readout_call(x1p, aggp, rootp, biasp, batch_row, fc0w, fc0b, fc1w, fc1b,
                  fc2w, fc2b):
  args = (x1p, aggp, rootp, biasp, batch_row, fc0w, fc0b, fc1w, fc1b, fc2w,
          fc2b)
  return pl.pallas_call(
      _readout_body,
      in_specs=[_full_spec(a.shape) for a in args],
      out_specs=_full_spec((N_GRAPHS, 1)),
      out_shape=jax.ShapeDtypeStruct((N_GRAPHS, 1), jnp.float32),
  )(*args)


_make_sc_gather = functools.lru_cache(maxsize=None)(_make_sc_gather)
_make_sc_scatter_add = functools.lru_cache(maxsize=None)(_make_sc_scatter_add)


@jax.jit
def kernel(x, edge_index, edge_attr, batch, nn0_w1, nn0_b1, nn0_w2, nn0_b2,
           root0, bias0, nn1_w1, nn1_b1, nn1_w2, nn1_b2, root1, bias1,
           fc0_w, fc0_b, fc1_w, fc1_b, fc2_w, fc2_b):
  pad_e = E_PAD - N_EDGES
  src = jnp.concatenate(
      [edge_index[0], jnp.zeros((pad_e,), jnp.int32)]).reshape(-1, CHUNK)
  dst = jnp.concatenate(
      [edge_index[1], jnp.full((pad_e,), N_NODES, jnp.int32)]).reshape(-1, CHUNK)
  ea = jnp.concatenate(
      [edge_attr, jnp.zeros((pad_e, B_IN), jnp.float32)], axis=0)
  w2cat0 = nn0_w2.reshape(F_IN, (DIM // 2) * F_IN)
  b2m0 = jnp.pad(nn0_b2.reshape(F_IN, DIM // 2), ((0, 0), (0, DIM // 2)))
  root1p = jnp.pad(root1, ((0, LANE - DIM // 2), (0, 0)))
  bias1p = bias1.reshape(1, DIM)
  b1r0 = nn0_b1.reshape(1, 128)
  b1r1 = nn1_b1.reshape(1, 128)
  b2r1 = nn1_b2.reshape(1, 128)
  batch_row = batch.reshape(1, N_NODES)

  sc_gather = _make_sc_gather()
  sc_scatter0 = _make_sc_scatter_add(8)
  sc_scatter1 = _make_sc_scatter_add(16)

  xg = sc_gather(src, x)
  msg0 = _msg0_call(xg, ea, nn0_w1, b1r0, w2cat0, b2m0)
  agg0 = _reduce_call(sc_scatter0(dst, msg0)).reshape(N_ACC, DIM // 2)
  x1p = _node0_call(x, agg0, root0, bias0.reshape(1, DIM // 2))

  x1g = sc_gather(src, x1p)
  msg1 = _msg1_call(x1g, ea, nn1_w1, b1r1, nn1_w2, b2r1)
  agg1 = _reduce_call(sc_scatter1(dst, msg1)).reshape(N_ACC, DIM)

  out = _readout_call(x1p, agg1, root1p, bias1p, batch_row,
                      fc0_w, fc0_b.reshape(1, -1),
                      fc1_w, fc1_b.reshape(1, -1),
                      fc2_w, fc2_b.reshape(1, -1))
  return out.reshape(-1)


# speedup vs baseline: 2.3601x; 1.4114x over previous
import functools

import jax
import jax.numpy as jnp
from jax import lax
from jax.experimental import pallas as pl
from jax.experimental.pallas import tpu as pltpu
from jax.experimental.pallas import tpu_sc as plsc

N_NODES = 10000
N_EDGES = 160000
F_IN = 128
B_IN = 16
DIM = 16
N_GRAPHS = 64
LANE = 128

NUM_CORES = 2
NUM_SUBCORES = 16
NW = NUM_CORES * NUM_SUBCORES
CHUNK = 128
CHUNKS_PER_W = 40
E_PER_W = CHUNK * CHUNKS_PER_W
E_PAD = NW * E_PER_W
N_ACC = 10240

BLK = 1024
N_EDGE_BLKS = E_PAD // BLK


def _make_sc_gather(out_w, from_spmem):
  mesh = plsc.VectorSubcoreMesh(core_axis_name="c", subcore_axis_name="s")
  rps = 640
  tail = N_NODES - rps * (NUM_SUBCORES - 1)

  nb = 2 if from_spmem else 4
  scratch = [
      pltpu.VMEM((CHUNKS_PER_W, CHUNK), jnp.int32),
      pltpu.VMEM((nb, CHUNK, LANE), jnp.float32),
      pltpu.SemaphoreType.DMA,
      pltpu.SemaphoreType.DMA,
      pltpu.SemaphoreType.DMA,
      pltpu.SemaphoreType.DMA,
      pltpu.SemaphoreType.DMA,
      pltpu.SemaphoreType.DMA,
      pltpu.SemaphoreType.DMA,
      pltpu.SemaphoreType.DMA,
  ]
  if from_spmem:
    scratch.append(pltpu.VMEM_SHARED((N_NODES, LANE), jnp.float32))

  @functools.partial(
      pl.kernel,
      mesh=mesh,
      out_type=jax.ShapeDtypeStruct((E_PAD, out_w), jnp.float32),
      scratch_types=scratch,
  )
  def gather_kernel(idx_hbm, table_hbm, out_hbm, idx_v, buf, g0, g1, g2, g3,
                    w0, w1, w2, w3, *maybe_shared):
    c = lax.axis_index("c")
    s = lax.axis_index("s")
    wid = s * NUM_CORES + c
    row0 = wid * CHUNKS_PER_W
    ebase = wid * E_PER_W
    gsems = (g0, g1, g2, g3)
    wsems = (w0, w1, w2, w3)
    if from_spmem:
      table = maybe_shared[0]

      @pl.when(s < NUM_SUBCORES - 1)
      def _():
        pltpu.sync_copy(table_hbm.at[pl.ds(s * rps, rps)],
                        table.at[pl.ds(s * rps, rps)])

      @pl.when(s == NUM_SUBCORES - 1)
      def _():
        pltpu.sync_copy(table_hbm.at[pl.ds((NUM_SUBCORES - 1) * rps, tail)],
                        table.at[pl.ds((NUM_SUBCORES - 1) * rps, tail)])

      pltpu.sync_copy(idx_hbm.at[pl.ds(row0, CHUNKS_PER_W)], idx_v)
      plsc.subcore_barrier()
    else:
      table = table_hbm
      pltpu.sync_copy(idx_hbm.at[pl.ds(row0, CHUNKS_PER_W)], idx_v)
    def src_buf(b):
      if out_w == LANE:
        return buf.at[b]
      return buf.at[b, :, pl.ds(0, out_w)]

    gd, wd = {}, {}
    for j in range(CHUNKS_PER_W):
      b = j % nb
      if j >= nb:
        wd[j - nb].wait()
      gd[j] = pltpu.async_copy(table.at[idx_v.at[j]], buf.at[b], gsems[b])
      if j >= 1:
        gd[j - 1].wait()
        wd[j - 1] = pltpu.async_copy(
            src_buf((j - 1) % nb),
            out_hbm.at[pl.ds(ebase + (j - 1) * CHUNK, CHUNK)],
            wsems[(j - 1) % nb])
    last = CHUNKS_PER_W - 1
    gd[last].wait()
    wd[last] = pltpu.async_copy(
        src_buf(last % nb), out_hbm.at[pl.ds(ebase + last * CHUNK, CHUNK)],
        wsems[last % nb])
    for j in range(CHUNKS_PER_W - nb, CHUNKS_PER_W):
      wd[j].wait()

  return gather_kernel


HALF = N_ACC // 2
ACC_W = HALF * DIM
RED = ACC_W // NUM_SUBCORES


def _make_sc_scatter_add(width):
  mesh = plsc.VectorSubcoreMesh(core_axis_name="c", subcore_axis_name="s")
  n_pass = 1 if width == 8 else 2
  rows = N_ACC if width == 8 else HALF

  @functools.partial(
      pl.kernel,
      mesh=mesh,
      out_type=jax.ShapeDtypeStruct((NW, n_pass * ACC_W), jnp.float32),
      compiler_params=pltpu.CompilerParams(needs_layout_passes=False),
      scratch_types=[
          pltpu.VMEM((CHUNKS_PER_W, CHUNK), jnp.int32),
          pltpu.VMEM((CHUNK, DIM), jnp.float32),
          pltpu.VMEM((CHUNK, DIM), jnp.float32),
          pltpu.VMEM((ACC_W,), jnp.float32),
          pltpu.SemaphoreType.DMA,
          pltpu.SemaphoreType.DMA,
      ],
  )
  def scatter_kernel(idx_hbm, msg_hbm, out_hbm, idx_v, m_a, m_b, acc,
                     sem_a, sem_b):
    c = lax.axis_index("c")
    s = lax.axis_index("s")
    wid = s * NUM_CORES + c
    ebase = wid * E_PER_W
    pltpu.sync_copy(idx_hbm.at[pl.ds(wid * CHUNKS_PER_W, CHUNKS_PER_W)], idx_v)
    lanes = lax.iota(jnp.int32, 16)
    zvec = jnp.zeros((16,), jnp.float32)
    const_mask = lanes < width

    def process_chunk(j, mb, lo):
      @pl.loop(0, CHUNK // 16)
      def _grp(g):
        dvec = idx_v[j, pl.ds(g * 16, 16)]
        for l in range(16):
          dd = dvec[l] - lo
          ddv = jnp.full((16,), dd, jnp.int32)
          if n_pass == 1:
            mask = const_mask
          else:
            mask = (ddv >= 0) & (ddv < rows)
          plsc.addupdate_scatter(acc, [ddv * width + lanes],
                                 mb[g * 16 + l, :], mask=mask)

    def start_load(j, buf, sem):
      return pltpu.async_copy(msg_hbm.at[pl.ds(ebase + j * CHUNK, CHUNK)],
                              buf, sem)

    for p in range(n_pass):
      lo = p * rows

      @pl.loop(0, ACC_W // 16, unroll=8)
      def _zero(k):
        acc[pl.ds(k * 16, 16)] = zvec

      start_load(0, m_a, sem_a)

      @pl.loop(0, CHUNKS_PER_W // 2)
      def _pair(t):
        j = t * 2
        start_load(j + 1, m_b, sem_b)
        pltpu.make_async_copy(msg_hbm.at[pl.ds(ebase, CHUNK)], m_a,
                              sem_a).wait()
        process_chunk(j, m_a, lo)

        @pl.when(t + 1 < CHUNKS_PER_W // 2)
        def _():
          start_load(j + 2, m_a, sem_a)

        pltpu.make_async_copy(msg_hbm.at[pl.ds(ebase, CHUNK)], m_b,
                              sem_b).wait()
        process_chunk(j + 1, m_b, lo)

      pltpu.sync_copy(acc, out_hbm.at[wid, pl.ds(p * ACC_W, ACC_W)])

  return scatter_kernel


def _reduce_body(parts_ref, out_ref):
  out_ref[...] = jnp.sum(parts_ref[...], axis=0, keepdims=True)


def _reduce_call(parts):
  blk = 8192
  total = parts.shape[1]
  return pl.pallas_call(
      _reduce_body,
      grid=(total // blk,),
      in_specs=[pl.BlockSpec((NW, blk), lambda i: (0, i))],
      out_specs=pl.BlockSpec((1, blk), lambda i: (0, i)),
      out_shape=jax.ShapeDtypeStruct((1, total), jnp.float32),
  )(parts)


def _edge_valid(blk):
  rows = lax.broadcasted_iota(jnp.int32, (blk, 1), 0) + pl.program_id(0) * blk
  return (rows < N_EDGES).astype(jnp.float32)


def _msg0_body(xg_ref, ea_ref, w1_ref, b1_ref, w2cat_ref, b2m_ref, out_ref):
  xg = xg_ref[...]
  ea = ea_ref[...]
  h = jnp.maximum(
      lax.dot_general(ea, w1_ref[...], (((1,), (1,)), ((), ()))) + b1_ref[...],
      0.0)
  cols = []
  for o in range(DIM // 2):
    p = jnp.dot(xg, w2cat_ref[:, o * 128:(o + 1) * 128],
                preferred_element_type=jnp.float32)
    cols.append(jnp.sum(p * h, axis=1, keepdims=True))
  zeros = jnp.zeros((xg.shape[0], DIM // 2), jnp.float32)
  msg = jnp.concatenate(cols + [zeros], axis=1) + jnp.dot(
      xg, b2m_ref[...], preferred_element_type=jnp.float32)
  out_ref[...] = msg * _edge_valid(xg.shape[0])


def _msg1_body(x1g_ref, ea_ref, w1_ref, b1_ref, w2_ref, b2_ref, out_ref):
  ea = ea_ref[...]
  h = jnp.maximum(
      lax.dot_general(ea, w1_ref[...], (((1,), (1,)), ((), ()))) + b1_ref[...],
      0.0)
  w1e = lax.dot_general(h, w2_ref[...], (((1,), (1,)), ((), ())))
  msg = jnp.zeros((ea.shape[0], DIM), jnp.float32)
  for i in range(DIM // 2):
    msg = msg + x1g_ref[:, i:i + 1] * (
        w1e[:, i * DIM:(i + 1) * DIM] + b2_ref[0:1, i * DIM:(i + 1) * DIM])
  out_ref[...] = msg * _edge_valid(ea.shape[0])


def _node0_body(x_ref, aggp_ref, root_ref, bias_ref, out_ref):
  agg = aggp_ref[:N_NODES, :]
  x1 = jnp.maximum(
      jnp.dot(x_ref[...], root_ref[...], preferred_element_type=jnp.float32)
      + agg + bias_ref[...], 0.0)
  out_ref[...] = jnp.concatenate(
      [x1, jnp.zeros((N_NODES, LANE - DIM // 2), jnp.float32)], axis=1)


def _readout_body(x1p_ref, aggp_ref, root_ref, bias_ref, batch_ref,
                  fc0w_ref, fc0b_ref, fc1w_ref, fc1b_ref, fc2w_ref, fc2b_ref,
                  out_ref):
  agg = aggp_ref[:N_NODES, :]
  x2 = jnp.maximum(
      jnp.dot(x1p_ref[...], root_ref[...], preferred_element_type=jnp.float32)
      + agg + bias_ref[...], 0.0)
  gids = lax.broadcasted_iota(jnp.int32, (N_GRAPHS, N_NODES), 0)
  onehot = (gids == batch_ref[...]).astype(jnp.float32)
  x2a = jnp.concatenate([x2, jnp.ones((N_NODES, 1), jnp.float32)], axis=1)
  seg = jnp.dot(onehot, x2a, preferred_element_type=jnp.float32)
  cnt = seg[:, DIM:DIM + 1]
  g = seg[:, :DIM] / jnp.maximum(cnt, 1.0)
  g = jnp.maximum(
      lax.dot_general(g, fc0w_ref[...], (((1,), (1,)), ((), ())))
      + fc0b_ref[...], 0.0)
  g = jnp.maximum(
      lax.dot_general(g, fc1w_ref[...], (((1,), (1,)), ((), ())))
      + fc1b_ref[...], 0.0)
  out_ref[...] = jnp.sum(g * fc2w_ref[...], axis=1, keepdims=True) + fc2b_ref[0, 0]


def _full_spec(shape):
  nd = len(shape)
  return pl.BlockSpec(shape, lambda i=0, *, _n=nd: (0,) * _n)


def _msg0_call(xg, ea, w1, b1, w2cat, b2m):
  in_specs = [pl.BlockSpec((BLK, LANE), lambda i: (i, 0)),
              pl.BlockSpec((BLK, B_IN), lambda i: (i, 0)),
              _full_spec(w1.shape), _full_spec(b1.shape),
              _full_spec(w2cat.shape), _full_spec(b2m.shape)]
  return pl.pallas_call(
      _msg0_body,
      grid=(N_EDGE_BLKS,),
      in_specs=in_specs,
      out_specs=pl.BlockSpec((BLK, DIM), lambda i: (i, 0)),
      out_shape=jax.ShapeDtypeStruct((E_PAD, DIM), jnp.float32),
  )(xg, ea, w1, b1, w2cat, b2m)


def _msg1_call(x1g, ea, w1, b1, w2, b2):
  in_specs = [pl.BlockSpec((BLK, LANE), lambda i: (i, 0)),
              pl.BlockSpec((BLK, B_IN), lambda i: (i, 0)),
              _full_spec(w1.shape), _full_spec(b1.shape),
              _full_spec(w2.shape), _full_spec(b2.shape)]
  return pl.pallas_call(
      _msg1_body,
      grid=(N_EDGE_BLKS,),
      in_specs=in_specs,
      out_specs=pl.BlockSpec((BLK, DIM), lambda i: (i, 0)),
      out_shape=jax.ShapeDtypeStruct((E_PAD, DIM), jnp.float32),
  )(x1g, ea, w1, b1, w2, b2)


def _node0_call(x, aggp, rootp, biasp):
  return pl.pallas_call(
      _node0_body,
      in_specs=[_full_spec(x.shape), _full_spec(aggp.shape),
                _full_spec(rootp.shape), _full_spec(biasp.shape)],
      out_specs=_full_spec((N_NODES, LANE)),
      out_shape=jax.ShapeDtypeStruct((N_NODES, LANE), jnp.float32),
  )(x, aggp, rootp, biasp)


def _readout_call(x1p, aggp, rootp, biasp, batch_row, fc0w, fc0b, fc1w, fc1b,
                  fc2w, fc2b):
  args = (x1p, aggp, rootp, biasp, batch_row, fc0w, fc0b, fc1w, fc1b, fc2w,
          fc2b)
  return pl.pallas_call(
      _readout_body,
      in_specs=[_full_spec(a.shape) for a in args],
      out_specs=_full_spec((N_GRAPHS, 1)),
      out_shape=jax.ShapeDtypeStruct((N_GRAPHS, 1), jnp.float32),
  )(*args)


_make_sc_gather = functools.lru_cache(maxsize=None)(_make_sc_gather)
_make_sc_scatter_add = functools.lru_cache(maxsize=None)(_make_sc_scatter_add)


@jax.jit
def kernel(x, edge_index, edge_attr, batch, nn0_w1, nn0_b1, nn0_w2, nn0_b2,
           root0, bias0, nn1_w1, nn1_b1, nn1_w2, nn1_b2, root1, bias1,
           fc0_w, fc0_b, fc1_w, fc1_b, fc2_w, fc2_b):
  pad_e = E_PAD - N_EDGES
  src = jnp.concatenate(
      [edge_index[0], jnp.zeros((pad_e,), jnp.int32)]).reshape(-1, CHUNK)
  dst = jnp.concatenate(
      [edge_index[1], jnp.full((pad_e,), N_NODES, jnp.int32)]).reshape(-1, CHUNK)
  ea = jnp.concatenate(
      [edge_attr, jnp.zeros((pad_e, B_IN), jnp.float32)], axis=0)
  w2cat0 = nn0_w2.reshape(F_IN, (DIM // 2) * F_IN)
  b2m0 = jnp.pad(nn0_b2.reshape(F_IN, DIM // 2), ((0, 0), (0, DIM // 2)))
  root1p = jnp.pad(root1, ((0, LANE - DIM // 2), (0, 0)))
  bias1p = bias1.reshape(1, DIM)
  b1r0 = nn0_b1.reshape(1, 128)
  b1r1 = nn1_b1.reshape(1, 128)
  b2r1 = nn1_b2.reshape(1, 128)
  batch_row = batch.reshape(1, N_NODES)

  sc_gather0 = _make_sc_gather(LANE, True)
  sc_gather1 = sc_gather0
  sc_scatter0 = _make_sc_scatter_add(8)
  sc_scatter1 = _make_sc_scatter_add(16)

  xg = sc_gather0(src, x)
  msg0 = _msg0_call(xg, ea, nn0_w1, b1r0, w2cat0, b2m0)
  agg0 = _reduce_call(sc_scatter0(dst, msg0)).reshape(N_ACC, DIM // 2)
  x1p = _node0_call(x, agg0, root0, bias0.reshape(1, DIM // 2))

  x1g = sc_gather1(src, x1p)
  msg1 = _msg1_call(x1g, ea, nn1_w1, b1r1, nn1_w2, b2r1)
  agg1 = _reduce_call(sc_scatter1(dst, msg1)).reshape(N_ACC, DIM)

  out = _readout_call(x1p, agg1, root1p, bias1p, batch_row,
                      fc0_w, fc0_b.reshape(1, -1),
                      fc1_w, fc1_b.reshape(1, -1),
                      fc2_w, fc2_b.reshape(1, -1))
  return out.reshape(-1)


# speedup vs baseline: 2.3658x; 1.0024x over previous
import functools

import jax
import jax.numpy as jnp
from jax import lax
from jax.experimental import pallas as pl
from jax.experimental.pallas import tpu as pltpu
from jax.experimental.pallas import tpu_sc as plsc

N_NODES = 10000
N_EDGES = 160000
F_IN = 128
B_IN = 16
DIM = 16
N_GRAPHS = 64
LANE = 128

NUM_CORES = 2
NUM_SUBCORES = 16
NW = NUM_CORES * NUM_SUBCORES
CHUNK = 128
CHUNKS_PER_W = 40
E_PER_W = CHUNK * CHUNKS_PER_W
E_PAD = NW * E_PER_W
N_ACC = 10240

BLK = 1024
N_EDGE_BLKS = E_PAD // BLK


def _make_sc_gather(out_w, from_spmem):
  mesh = plsc.VectorSubcoreMesh(core_axis_name="c", subcore_axis_name="s")
  rps = 640
  tail = N_NODES - rps * (NUM_SUBCORES - 1)

  nb = 2 if from_spmem else 4
  scratch = [
      pltpu.VMEM((CHUNKS_PER_W, CHUNK), jnp.int32),
      pltpu.VMEM((nb, CHUNK, LANE), jnp.float32),
      pltpu.SemaphoreType.DMA,
      pltpu.SemaphoreType.DMA,
      pltpu.SemaphoreType.DMA,
      pltpu.SemaphoreType.DMA,
      pltpu.SemaphoreType.DMA,
      pltpu.SemaphoreType.DMA,
      pltpu.SemaphoreType.DMA,
      pltpu.SemaphoreType.DMA,
  ]
  if from_spmem:
    scratch.append(pltpu.VMEM_SHARED((N_NODES, LANE), jnp.float32))

  @functools.partial(
      pl.kernel,
      mesh=mesh,
      out_type=jax.ShapeDtypeStruct((E_PAD, out_w), jnp.float32),
      scratch_types=scratch,
  )
  def gather_kernel(idx_hbm, table_hbm, out_hbm, idx_v, buf, g0, g1, g2, g3,
                    w0, w1, w2, w3, *maybe_shared):
    c = lax.axis_index("c")
    s = lax.axis_index("s")
    wid = s * NUM_CORES + c
    row0 = wid * CHUNKS_PER_W
    ebase = wid * E_PER_W
    gsems = (g0, g1, g2, g3)
    wsems = (w0, w1, w2, w3)
    if from_spmem:
      table = maybe_shared[0]

      @pl.when(s < NUM_SUBCORES - 1)
      def _():
        pltpu.sync_copy(table_hbm.at[pl.ds(s * rps, rps)],
                        table.at[pl.ds(s * rps, rps)])

      @pl.when(s == NUM_SUBCORES - 1)
      def _():
        pltpu.sync_copy(table_hbm.at[pl.ds((NUM_SUBCORES - 1) * rps, tail)],
                        table.at[pl.ds((NUM_SUBCORES - 1) * rps, tail)])

      pltpu.sync_copy(idx_hbm.at[pl.ds(row0, CHUNKS_PER_W)], idx_v)
      plsc.subcore_barrier()
    else:
      table = table_hbm
      pltpu.sync_copy(idx_hbm.at[pl.ds(row0, CHUNKS_PER_W)], idx_v)
    def src_buf(b):
      if out_w == LANE:
        return buf.at[b]
      return buf.at[b, :, pl.ds(0, out_w)]

    gd, wd = {}, {}
    for j in range(CHUNKS_PER_W):
      b = j % nb
      if j >= nb:
        wd[j - nb].wait()
      gd[j] = pltpu.async_copy(table.at[idx_v.at[j]], buf.at[b], gsems[b])
      if j >= 1:
        gd[j - 1].wait()
        wd[j - 1] = pltpu.async_copy(
            src_buf((j - 1) % nb),
            out_hbm.at[pl.ds(ebase + (j - 1) * CHUNK, CHUNK)],
            wsems[(j - 1) % nb])
    last = CHUNKS_PER_W - 1
    gd[last].wait()
    wd[last] = pltpu.async_copy(
        src_buf(last % nb), out_hbm.at[pl.ds(ebase + last * CHUNK, CHUNK)],
        wsems[last % nb])
    for j in range(CHUNKS_PER_W - nb, CHUNKS_PER_W):
      wd[j].wait()

  return gather_kernel


HALF = N_ACC // 2
ACC_W = HALF * DIM
RED = ACC_W // NUM_SUBCORES


def _make_sc_scatter_add(width):
  mesh = plsc.VectorSubcoreMesh(core_axis_name="c", subcore_axis_name="s")
  n_pass = 1 if width == 8 else 2
  rows = N_ACC if width == 8 else HALF

  @functools.partial(
      pl.kernel,
      mesh=mesh,
      out_type=jax.ShapeDtypeStruct((NW, n_pass * ACC_W), jnp.float32),
      compiler_params=pltpu.CompilerParams(needs_layout_passes=False),
      scratch_types=[
          pltpu.VMEM((CHUNKS_PER_W, CHUNK), jnp.int32),
          pltpu.VMEM((CHUNK, DIM), jnp.float32),
          pltpu.VMEM((CHUNK, DIM), jnp.float32),
          pltpu.VMEM((ACC_W,), jnp.float32),
          pltpu.SemaphoreType.DMA,
          pltpu.SemaphoreType.DMA,
      ],
  )
  def scatter_kernel(idx_hbm, msg_hbm, out_hbm, idx_v, m_a, m_b, acc,
                     sem_a, sem_b):
    c = lax.axis_index("c")
    s = lax.axis_index("s")
    wid = s * NUM_CORES + c
    ebase = wid * E_PER_W
    pltpu.sync_copy(idx_hbm.at[pl.ds(wid * CHUNKS_PER_W, CHUNKS_PER_W)], idx_v)
    lanes = lax.iota(jnp.int32, 16)
    zvec = jnp.zeros((16,), jnp.float32)
    const_mask = lanes < width

    def process_chunk(j, mb, lo):
      @pl.loop(0, CHUNK // 16)
      def _grp(g):
        dvec = idx_v[j, pl.ds(g * 16, 16)]
        for l in range(16):
          dd = dvec[l] - lo
          ddv = jnp.full((16,), dd, jnp.int32)
          if n_pass == 1:
            mask = const_mask
          else:
            mask = (ddv >= 0) & (ddv < rows)
          plsc.addupdate_scatter(acc, [ddv * width + lanes],
                                 mb[g * 16 + l, :], mask=mask)

    def start_load(j, buf, sem):
      return pltpu.async_copy(msg_hbm.at[pl.ds(ebase + j * CHUNK, CHUNK)],
                              buf, sem)

    for p in range(n_pass):
      lo = p * rows

      @pl.loop(0, ACC_W // 16, unroll=8)
      def _zero(k):
        acc[pl.ds(k * 16, 16)] = zvec

      start_load(0, m_a, sem_a)

      @pl.loop(0, CHUNKS_PER_W // 2)
      def _pair(t):
        j = t * 2
        start_load(j + 1, m_b, sem_b)
        pltpu.make_async_copy(msg_hbm.at[pl.ds(ebase, CHUNK)], m_a,
                              sem_a).wait()
        process_chunk(j, m_a, lo)

        @pl.when(t + 1 < CHUNKS_PER_W // 2)
        def _():
          start_load(j + 2, m_a, sem_a)

        pltpu.make_async_copy(msg_hbm.at[pl.ds(ebase, CHUNK)], m_b,
                              sem_b).wait()
        process_chunk(j + 1, m_b, lo)

      pltpu.sync_copy(acc, out_hbm.at[wid, pl.ds(p * ACC_W, ACC_W)])

  return scatter_kernel


def _reduce_body(parts_ref, out_ref):
  out_ref[...] = jnp.sum(parts_ref[...], axis=0, keepdims=True)


def _reduce_call(parts):
  blk = 8192
  total = parts.shape[1]
  return pl.pallas_call(
      _reduce_body,
      grid=(total // blk,),
      in_specs=[pl.BlockSpec((NW, blk), lambda i: (0, i))],
      out_specs=pl.BlockSpec((1, blk), lambda i: (0, i)),
      out_shape=jax.ShapeDtypeStruct((1, total), jnp.float32),
  )(parts)


def _edge_valid(blk):
  rows = lax.broadcasted_iota(jnp.int32, (blk, 1), 0) + pl.program_id(0) * blk
  return (rows < N_EDGES).astype(jnp.float32)


def _msg0_body(xg_ref, ea_ref, w1_ref, b1_ref, w2cat_ref, b2m_ref, out_ref):
  xg = xg_ref[...]
  ea = ea_ref[...]
  h = jnp.maximum(
      lax.dot_general(ea, w1_ref[...], (((1,), (1,)), ((), ()))) + b1_ref[...],
      0.0)
  xgb = xg.astype(jnp.bfloat16)
  cols = []
  for o in range(DIM // 2):
    p = jnp.dot(xgb, w2cat_ref[:, o * 128:(o + 1) * 128].astype(jnp.bfloat16),
                preferred_element_type=jnp.float32)
    cols.append(jnp.sum(p * h, axis=1, keepdims=True))
  zeros = jnp.zeros((xg.shape[0], DIM // 2), jnp.float32)
  msg = jnp.concatenate(cols + [zeros], axis=1) + jnp.dot(
      xg, b2m_ref[...], preferred_element_type=jnp.float32)
  out_ref[...] = msg * _edge_valid(xg.shape[0])


def _msg1_body(x1g_ref, ea_ref, w1_ref, b1_ref, w2_ref, b2_ref, out_ref):
  ea = ea_ref[...]
  h = jnp.maximum(
      lax.dot_general(ea, w1_ref[...], (((1,), (1,)), ((), ()))) + b1_ref[...],
      0.0)
  w1e = lax.dot_general(h.astype(jnp.bfloat16),
                        w2_ref[...].astype(jnp.bfloat16),
                        (((1,), (1,)), ((), ())),
                        preferred_element_type=jnp.float32)
  msg = jnp.zeros((ea.shape[0], DIM), jnp.float32)
  for i in range(DIM // 2):
    msg = msg + x1g_ref[:, i:i + 1] * (
        w1e[:, i * DIM:(i + 1) * DIM] + b2_ref[0:1, i * DIM:(i + 1) * DIM])
  out_ref[...] = msg * _edge_valid(ea.shape[0])


def _node0_body(x_ref, aggp_ref, root_ref, bias_ref, out_ref):
  agg = aggp_ref[:N_NODES, :]
  x1 = jnp.maximum(
      jnp.dot(x_ref[...], root_ref[...], preferred_element_type=jnp.float32)
      + agg + bias_ref[...], 0.0)
  out_ref[...] = jnp.concatenate(
      [x1, jnp.zeros((N_NODES, LANE - DIM // 2), jnp.float32)], axis=1)


def _readout_body(x1p_ref, aggp_ref, root_ref, bias_ref, batch_ref,
                  fc0w_ref, fc0b_ref, fc1w_ref, fc1b_ref, fc2w_ref, fc2b_ref,
                  out_ref):
  agg = aggp_ref[:N_NODES, :]
  x2 = jnp.maximum(
      jnp.dot(x1p_ref[...], root_ref[...], preferred_element_type=jnp.float32)
      + agg + bias_ref[...], 0.0)
  gids = lax.broadcasted_iota(jnp.int32, (N_GRAPHS, N_NODES), 0)
  onehot = (gids == batch_ref[...]).astype(jnp.float32)
  x2a = jnp.concatenate([x2, jnp.ones((N_NODES, 1), jnp.float32)], axis=1)
  seg = jnp.dot(onehot, x2a, preferred_element_type=jnp.float32)
  cnt = seg[:, DIM:DIM + 1]
  g = seg[:, :DIM] / jnp.maximum(cnt, 1.0)
  g = jnp.maximum(
      lax.dot_general(g, fc0w_ref[...], (((1,), (1,)), ((), ())))
      + fc0b_ref[...], 0.0)
  g = jnp.maximum(
      lax.dot_general(g, fc1w_ref[...], (((1,), (1,)), ((), ())))
      + fc1b_ref[...], 0.0)
  out_ref[...] = jnp.sum(g * fc2w_ref[...], axis=1, keepdims=True) + fc2b_ref[0, 0]


def _full_spec(shape):
  nd = len(shape)
  return pl.BlockSpec(shape, lambda i=0, *, _n=nd: (0,) * _n)


def _msg0_call(xg, ea, w1, b1, w2cat, b2m):
  in_specs = [pl.BlockSpec((BLK, LANE), lambda i: (i, 0)),
              pl.BlockSpec((BLK, B_IN), lambda i: (i, 0)),
              _full_spec(w1.shape), _full_spec(b1.shape),
              _full_spec(w2cat.shape), _full_spec(b2m.shape)]
  return pl.pallas_call(
      _msg0_body,
      grid=(N_EDGE_BLKS,),
      in_specs=in_specs,
      out_specs=pl.BlockSpec((BLK, DIM), lambda i: (i, 0)),
      out_shape=jax.ShapeDtypeStruct((E_PAD, DIM), jnp.float32),
  )(xg, ea, w1, b1, w2cat, b2m)


def _msg1_call(x1g, ea, w1, b1, w2, b2):
  in_specs = [pl.BlockSpec((BLK, LANE), lambda i: (i, 0)),
              pl.BlockSpec((BLK, B_IN), lambda i: (i, 0)),
              _full_spec(w1.shape), _full_spec(b1.shape),
              _full_spec(w2.shape), _full_spec(b2.shape)]
  return pl.pallas_call(
      _msg1_body,
      grid=(N_EDGE_BLKS,),
      in_specs=in_specs,
      out_specs=pl.BlockSpec((BLK, DIM), lambda i: (i, 0)),
      out_shape=jax.ShapeDtypeStruct((E_PAD, DIM), jnp.float32),
  )(x1g, ea, w1, b1, w2, b2)


def _node0_call(x, aggp, rootp, biasp):
  return pl.pallas_call(
      _node0_body,
      in_specs=[_full_spec(x.shape), _full_spec(aggp.shape),
                _full_spec(rootp.shape), _full_spec(biasp.shape)],
      out_specs=_full_spec((N_NODES, LANE)),
      out_shape=jax.ShapeDtypeStruct((N_NODES, LANE), jnp.float32),
  )(x, aggp, rootp, biasp)


def _readout_call(x1p, aggp, rootp, biasp, batch_row, fc0w, fc0b, fc1w, fc1b,
                  fc2w, fc2b):
  args = (x1p, aggp, rootp, biasp, batch_row, fc0w, fc0b, fc1w, fc1b, fc2w,
          fc2b)
  return pl.pallas_call(
      _readout_body,
      in_specs=[_full_spec(a.shape) for a in args],
      out_specs=_full_spec((N_GRAPHS, 1)),
      out_shape=jax.ShapeDtypeStruct((N_GRAPHS, 1), jnp.float32),
  )(*args)


_make_sc_gather = functools.lru_cache(maxsize=None)(_make_sc_gather)
_make_sc_scatter_add = functools.lru_cache(maxsize=None)(_make_sc_scatter_add)


@jax.jit
def kernel(x, edge_index, edge_attr, batch, nn0_w1, nn0_b1, nn0_w2, nn0_b2,
           root0, bias0, nn1_w1, nn1_b1, nn1_w2, nn1_b2, root1, bias1,
           fc0_w, fc0_b, fc1_w, fc1_b, fc2_w, fc2_b):
  pad_e = E_PAD - N_EDGES
  src = jnp.concatenate(
      [edge_index[0], jnp.zeros((pad_e,), jnp.int32)]).reshape(-1, CHUNK)
  dst = jnp.concatenate(
      [edge_index[1], jnp.full((pad_e,), N_NODES, jnp.int32)]).reshape(-1, CHUNK)
  ea = jnp.concatenate(
      [edge_attr, jnp.zeros((pad_e, B_IN), jnp.float32)], axis=0)
  w2cat0 = nn0_w2.reshape(F_IN, (DIM // 2) * F_IN)
  b2m0 = jnp.pad(nn0_b2.reshape(F_IN, DIM // 2), ((0, 0), (0, DIM // 2)))
  root1p = jnp.pad(root1, ((0, LANE - DIM // 2), (0, 0)))
  bias1p = bias1.reshape(1, DIM)
  b1r0 = nn0_b1.reshape(1, 128)
  b1r1 = nn1_b1.reshape(1, 128)
  b2r1 = nn1_b2.reshape(1, 128)
  batch_row = batch.reshape(1, N_NODES)

  sc_gather0 = _make_sc_gather(LANE, True)
  sc_gather1 = sc_gather0
  sc_scatter0 = _make_sc_scatter_add(8)
  sc_scatter1 = _make_sc_scatter_add(16)

  xg = sc_gather0(src, x)
  msg0 = _msg0_call(xg, ea, nn0_w1, b1r0, w2cat0, b2m0)
  agg0 = _reduce_call(sc_scatter0(dst, msg0)).reshape(N_ACC, DIM // 2)
  x1p = _node0_call(x, agg0, root0, bias0.reshape(1, DIM // 2))

  x1g = sc_gather1(src, x1p)
  msg1 = _msg1_call(x1g, ea, nn1_w1, b1r1, nn1_w2, b2r1)
  agg1 = _reduce_call(sc_scatter1(dst, msg1)).reshape(N_ACC, DIM)

  out = _readout_call(x1p, agg1, root1p, bias1p, batch_row,
                      fc0_w, fc0_b.reshape(1, -1),
                      fc1_w, fc1_b.reshape(1, -1),
                      fc2_w, fc2_b.reshape(1, -1))
  return out.reshape(-1)


# speedup vs baseline: 2.5402x; 1.0737x over previous
import functools

import jax
import jax.numpy as jnp
from jax import lax
from jax.experimental import pallas as pl
from jax.experimental.pallas import tpu as pltpu
from jax.experimental.pallas import tpu_sc as plsc

N_NODES = 10000
N_EDGES = 160000
F_IN = 128
B_IN = 16
DIM = 16
N_GRAPHS = 64
LANE = 128

NUM_CORES = 2
NUM_SUBCORES = 16
NW = NUM_CORES * NUM_SUBCORES
CHUNK = 128
CHUNKS_PER_W = 40
E_PER_W = CHUNK * CHUNKS_PER_W
E_PAD = NW * E_PER_W
N_ACC = 10240

BLK = 1024
N_EDGE_BLKS = E_PAD // BLK

NCH = CHUNKS_PER_W // 2
E_PER_W_H = NCH * CHUNK
E_HALF = E_PAD // 2
N_EDGE_BLKS_H = E_HALF // BLK
IDX_ROWS_H = E_HALF // CHUNK


def _make_sc_gather(out_w, from_spmem, half):
  mesh = plsc.VectorSubcoreMesh(core_axis_name="c", subcore_axis_name="s")
  rps = 640
  tail = N_NODES - rps * (NUM_SUBCORES - 1)

  nb = 2 if from_spmem else 4
  scratch = [
      pltpu.VMEM((NCH + 4, CHUNK), jnp.int32),
      pltpu.VMEM((nb, CHUNK, LANE), jnp.float32),
      pltpu.SemaphoreType.DMA,
      pltpu.SemaphoreType.DMA,
      pltpu.SemaphoreType.DMA,
      pltpu.SemaphoreType.DMA,
      pltpu.SemaphoreType.DMA,
      pltpu.SemaphoreType.DMA,
      pltpu.SemaphoreType.DMA,
      pltpu.SemaphoreType.DMA,
  ]
  if from_spmem:
    scratch.append(pltpu.VMEM_SHARED((N_NODES, LANE), jnp.float32))

  @functools.partial(
      pl.kernel,
      mesh=mesh,
      out_type=jax.ShapeDtypeStruct((E_HALF, out_w), jnp.float32),
      scratch_types=scratch,
  )
  def gather_kernel(idx_hbm, table_hbm, out_hbm, idx_v, buf, g0, g1, g2, g3,
                    w0, w1, w2, w3, *maybe_shared):
    c = lax.axis_index("c")
    s = lax.axis_index("s")
    wid = s * NUM_CORES + c
    off = (wid % 2) * 4
    row0 = pl.multiple_of(half * IDX_ROWS_H + wid * NCH - off, 8)
    ebase = wid * E_PER_W_H
    gsems = (g0, g1, g2, g3)
    wsems = (w0, w1, w2, w3)
    if from_spmem:
      table = maybe_shared[0]

      @pl.when(s < NUM_SUBCORES - 1)
      def _():
        pltpu.sync_copy(table_hbm.at[pl.ds(s * rps, rps)],
                        table.at[pl.ds(s * rps, rps)])

      @pl.when(s == NUM_SUBCORES - 1)
      def _():
        pltpu.sync_copy(table_hbm.at[pl.ds((NUM_SUBCORES - 1) * rps, tail)],
                        table.at[pl.ds((NUM_SUBCORES - 1) * rps, tail)])

      pltpu.sync_copy(idx_hbm.at[pl.ds(row0, NCH + 4)], idx_v)
      plsc.subcore_barrier()
    else:
      table = table_hbm
      pltpu.sync_copy(idx_hbm.at[pl.ds(row0, NCH + 4)], idx_v)
    def src_buf(b):
      if out_w == LANE:
        return buf.at[b]
      return buf.at[b, :, pl.ds(0, out_w)]

    gd, wd = {}, {}
    for j in range(NCH):
      b = j % nb
      if j >= nb:
        wd[j - nb].wait()
      gd[j] = pltpu.async_copy(table.at[idx_v.at[off + j]], buf.at[b],
                               gsems[b])
      if j >= 1:
        gd[j - 1].wait()
        wd[j - 1] = pltpu.async_copy(
            src_buf((j - 1) % nb),
            out_hbm.at[pl.ds(ebase + (j - 1) * CHUNK, CHUNK)],
            wsems[(j - 1) % nb])
    last = NCH - 1
    gd[last].wait()
    wd[last] = pltpu.async_copy(
        src_buf(last % nb), out_hbm.at[pl.ds(ebase + last * CHUNK, CHUNK)],
        wsems[last % nb])
    for j in range(NCH - nb, NCH):
      wd[j].wait()

  return gather_kernel


HALF = N_ACC // 2
ACC_W = HALF * DIM
RED = ACC_W // NUM_SUBCORES


def _make_sc_scatter_add(width, half):
  mesh = plsc.VectorSubcoreMesh(core_axis_name="c", subcore_axis_name="s")
  n_pass = 1 if width == 8 else 2
  rows = N_ACC if width == 8 else HALF

  @functools.partial(
      pl.kernel,
      mesh=mesh,
      out_type=jax.ShapeDtypeStruct((NW, n_pass * ACC_W), jnp.float32),
      compiler_params=pltpu.CompilerParams(needs_layout_passes=False),
      scratch_types=[
          pltpu.VMEM((NCH + 4, CHUNK), jnp.int32),
          pltpu.VMEM((CHUNK, DIM), jnp.float32),
          pltpu.VMEM((CHUNK, DIM), jnp.float32),
          pltpu.VMEM((ACC_W,), jnp.float32),
          pltpu.SemaphoreType.DMA,
          pltpu.SemaphoreType.DMA,
      ],
  )
  def scatter_kernel(idx_hbm, msg_hbm, out_hbm, idx_v, m_a, m_b, acc,
                     sem_a, sem_b):
    c = lax.axis_index("c")
    s = lax.axis_index("s")
    wid = s * NUM_CORES + c
    off = (wid % 2) * 4
    ebase = wid * E_PER_W_H
    pltpu.sync_copy(
        idx_hbm.at[pl.ds(
            pl.multiple_of(half * IDX_ROWS_H + wid * NCH - off, 8), NCH + 4)],
        idx_v)
    lanes = lax.iota(jnp.int32, 16)
    zvec = jnp.zeros((16,), jnp.float32)
    const_mask = lanes < width

    def process_chunk(j, mb, lo):
      @pl.loop(0, CHUNK // 16)
      def _grp(g):
        dvec = idx_v[off + j, pl.ds(g * 16, 16)]
        for l in range(16):
          dd = dvec[l] - lo
          ddv = jnp.full((16,), dd, jnp.int32)
          if n_pass == 1:
            mask = const_mask
          else:
            mask = (ddv >= 0) & (ddv < rows)
          plsc.addupdate_scatter(acc, [ddv * width + lanes],
                                 mb[g * 16 + l, :], mask=mask)

    def start_load(j, buf, sem):
      return pltpu.async_copy(msg_hbm.at[pl.ds(ebase + j * CHUNK, CHUNK)],
                              buf, sem)

    for p in range(n_pass):
      lo = p * rows

      @pl.loop(0, ACC_W // 16, unroll=8)
      def _zero(k):
        acc[pl.ds(k * 16, 16)] = zvec

      start_load(0, m_a, sem_a)

      @pl.loop(0, NCH // 2)
      def _pair(t):
        j = t * 2
        start_load(j + 1, m_b, sem_b)
        pltpu.make_async_copy(msg_hbm.at[pl.ds(ebase, CHUNK)], m_a,
                              sem_a).wait()
        process_chunk(j, m_a, lo)

        @pl.when(t + 1 < NCH // 2)
        def _():
          start_load(j + 2, m_a, sem_a)

        pltpu.make_async_copy(msg_hbm.at[pl.ds(ebase, CHUNK)], m_b,
                              sem_b).wait()
        process_chunk(j + 1, m_b, lo)

      pltpu.sync_copy(acc, out_hbm.at[wid, pl.ds(p * ACC_W, ACC_W)])

  return scatter_kernel


def _reduce_body(pa_ref, pb_ref, out_ref):
  out_ref[...] = (jnp.sum(pa_ref[...], axis=0, keepdims=True)
                  + jnp.sum(pb_ref[...], axis=0, keepdims=True))


def _reduce_call(parts_a, parts_b, width):
  blk = 8192
  total = parts_a.shape[1]
  flat = pl.pallas_call(
      _reduce_body,
      grid=(total // blk,),
      in_specs=[pl.BlockSpec((NW, blk), lambda i: (0, i)),
                pl.BlockSpec((NW, blk), lambda i: (0, i))],
      out_specs=pl.BlockSpec((1, blk), lambda i: (0, i)),
      out_shape=jax.ShapeDtypeStruct((1, total), jnp.float32),
  )(parts_a, parts_b)
  return flat.reshape(total // width, width)


def _edge_valid(blk, half):
  rows = (lax.broadcasted_iota(jnp.int32, (blk, 1), 0)
          + (pl.program_id(0) + half * N_EDGE_BLKS_H) * blk)
  return (rows < N_EDGES).astype(jnp.float32)


def _make_msg0_body(half):
  def _msg0_body(xg_ref, ea_ref, w1_ref, b1_ref, w2cat_ref, b2m_ref, out_ref):
    xg = xg_ref[...]
    ea = ea_ref[...]
    h = jnp.maximum(
        lax.dot_general(ea, w1_ref[...], (((1,), (1,)), ((), ())))
        + b1_ref[...], 0.0)
    cols = []
    for o in range(DIM // 2):
      p = jnp.dot(xg, w2cat_ref[:, o * 128:(o + 1) * 128],
                  preferred_element_type=jnp.float32)
      cols.append(jnp.sum(p * h, axis=1, keepdims=True))
    zeros = jnp.zeros((xg.shape[0], DIM // 2), jnp.float32)
    msg = jnp.concatenate(cols + [zeros], axis=1) + jnp.dot(
        xg, b2m_ref[...], preferred_element_type=jnp.float32)
    out_ref[...] = msg * _edge_valid(xg.shape[0], half)
  return _msg0_body


def _make_msg1_body(half):
  def _msg1_body(x1g_ref, ea_ref, w1_ref, b1_ref, w2_ref, b2_ref, out_ref):
    ea = ea_ref[...]
    h = jnp.maximum(
        lax.dot_general(ea, w1_ref[...], (((1,), (1,)), ((), ())))
        + b1_ref[...], 0.0)
    w1e = lax.dot_general(h, w2_ref[...], (((1,), (1,)), ((), ())))
    msg = jnp.zeros((ea.shape[0], DIM), jnp.float32)
    for i in range(DIM // 2):
      msg = msg + x1g_ref[:, i:i + 1] * (
          w1e[:, i * DIM:(i + 1) * DIM] + b2_ref[0:1, i * DIM:(i + 1) * DIM])
    out_ref[...] = msg * _edge_valid(ea.shape[0], half)
  return _msg1_body


def _node0_body(x_ref, aggp_ref, root_ref, bias_ref, out_ref):
  agg = aggp_ref[:N_NODES, :]
  x1 = jnp.maximum(
      jnp.dot(x_ref[...], root_ref[...], preferred_element_type=jnp.float32)
      + agg + bias_ref[...], 0.0)
  out_ref[...] = jnp.concatenate(
      [x1, jnp.zeros((N_NODES, LANE - DIM // 2), jnp.float32)], axis=1)


def _readout_body(x1p_ref, aggp_ref, root_ref, bias_ref, batch_ref,
                  fc0w_ref, fc0b_ref, fc1w_ref, fc1b_ref, fc2w_ref, fc2b_ref,
                  out_ref):
  agg = aggp_ref[:N_NODES, :]
  x2 = jnp.maximum(
      jnp.dot(x1p_ref[...], root_ref[...], preferred_element_type=jnp.float32)
      + agg + bias_ref[...], 0.0)
  gids = lax.broadcasted_iota(jnp.int32, (N_GRAPHS, N_NODES), 0)
  onehot = (gids == batch_ref[...]).astype(jnp.float32)
  x2a = jnp.concatenate([x2, jnp.ones((N_NODES, 1), jnp.float32)], axis=1)
  seg = jnp.dot(onehot, x2a, preferred_element_type=jnp.float32)
  cnt = seg[:, DIM:DIM + 1]
  g = seg[:, :DIM] / jnp.maximum(cnt, 1.0)
  g = jnp.maximum(
      lax.dot_general(g, fc0w_ref[...], (((1,), (1,)), ((), ())))
      + fc0b_ref[...], 0.0)
  g = jnp.maximum(
      lax.dot_general(g, fc1w_ref[...], (((1,), (1,)), ((), ())))
      + fc1b_ref[...], 0.0)
  out_ref[...] = jnp.sum(g * fc2w_ref[...], axis=1, keepdims=True) + fc2b_ref[0, 0]


def _full_spec(shape):
  nd = len(shape)
  return pl.BlockSpec(shape, lambda i=0, *, _n=nd: (0,) * _n)


def _msg0_call(xg, ea, w1, b1, w2cat, b2m, half):
  in_specs = [pl.BlockSpec((BLK, LANE), lambda i: (i, 0)),
              pl.BlockSpec((BLK, B_IN),
                           lambda i, _h=half: (i + _h * N_EDGE_BLKS_H, 0)),
              _full_spec(w1.shape), _full_spec(b1.shape),
              _full_spec(w2cat.shape), _full_spec(b2m.shape)]
  return pl.pallas_call(
      _make_msg0_body(half),
      grid=(N_EDGE_BLKS_H,),
      in_specs=in_specs,
      out_specs=pl.BlockSpec((BLK, DIM), lambda i: (i, 0)),
      out_shape=jax.ShapeDtypeStruct((E_HALF, DIM), jnp.float32),
  )(xg, ea, w1, b1, w2cat, b2m)


def _msg1_call(x1g, ea, w1, b1, w2, b2, half):
  in_specs = [pl.BlockSpec((BLK, LANE), lambda i: (i, 0)),
              pl.BlockSpec((BLK, B_IN),
                           lambda i, _h=half: (i + _h * N_EDGE_BLKS_H, 0)),
              _full_spec(w1.shape), _full_spec(b1.shape),
              _full_spec(w2.shape), _full_spec(b2.shape)]
  return pl.pallas_call(
      _make_msg1_body(half),
      grid=(N_EDGE_BLKS_H,),
      in_specs=in_specs,
      out_specs=pl.BlockSpec((BLK, DIM), lambda i: (i, 0)),
      out_shape=jax.ShapeDtypeStruct((E_HALF, DIM), jnp.float32),
  )(x1g, ea, w1, b1, w2, b2)


def _node0_call(x, aggp, rootp, biasp):
  return pl.pallas_call(
      _node0_body,
      in_specs=[_full_spec(x.shape), _full_spec(aggp.shape),
                _full_spec(rootp.shape), _full_spec(biasp.shape)],
      out_specs=_full_spec((N_NODES, LANE)),
      out_shape=jax.ShapeDtypeStruct((N_NODES, LANE), jnp.float32),
  )(x, aggp, rootp, biasp)


def _readout_call(x1p, aggp, rootp, biasp, batch_row, fc0w, fc0b, fc1w, fc1b,
                  fc2w, fc2b):
  args = (x1p, aggp, rootp, biasp, batch_row, fc0w, fc0b, fc1w, fc1b, fc2w,
          fc2b)
  return pl.pallas_call(
      _readout_body,
      in_specs=[_full_spec(a.shape) for a in args],
      out_specs=_full_spec((N_GRAPHS, 1)),
      out_shape=jax.ShapeDtypeStruct((N_GRAPHS, 1), jnp.float32),
  )(*args)


_make_sc_gather = functools.lru_cache(maxsize=None)(_make_sc_gather)
_make_sc_scatter_add = functools.lru_cache(maxsize=None)(_make_sc_scatter_add)


@jax.jit
def kernel(x, edge_index, edge_attr, batch, nn0_w1, nn0_b1, nn0_w2, nn0_b2,
           root0, bias0, nn1_w1, nn1_b1, nn1_w2, nn1_b2, root1, bias1,
           fc0_w, fc0_b, fc1_w, fc1_b, fc2_w, fc2_b):
  pad_e = E_PAD - N_EDGES
  src = jnp.concatenate(
      [edge_index[0], jnp.zeros((pad_e,), jnp.int32)]).reshape(-1, CHUNK)
  dst = jnp.concatenate(
      [edge_index[1], jnp.full((pad_e,), N_NODES, jnp.int32)]).reshape(-1, CHUNK)
  ea = jnp.concatenate(
      [edge_attr, jnp.zeros((pad_e, B_IN), jnp.float32)], axis=0)
  w2cat0 = nn0_w2.reshape(F_IN, (DIM // 2) * F_IN)
  b2m0 = jnp.pad(nn0_b2.reshape(F_IN, DIM // 2), ((0, 0), (0, DIM // 2)))
  root1p = jnp.pad(root1, ((0, LANE - DIM // 2), (0, 0)))
  bias1p = bias1.reshape(1, DIM)
  b1r0 = nn0_b1.reshape(1, 128)
  b1r1 = nn1_b1.reshape(1, 128)
  b2r1 = nn1_b2.reshape(1, 128)
  batch_row = batch.reshape(1, N_NODES)

  gat = [_make_sc_gather(LANE, True, h) for h in (0, 1)]
  sc0 = [_make_sc_scatter_add(8, h) for h in (0, 1)]
  sc1 = [_make_sc_scatter_add(16, h) for h in (0, 1)]

  xg = [gat[h](src, x) for h in (0, 1)]
  msg0 = [_msg0_call(xg[h], ea, nn0_w1, b1r0, w2cat0, b2m0, h) for h in (0, 1)]
  parts0 = [sc0[h](dst, msg0[h]) for h in (0, 1)]
  agg0 = _reduce_call(parts0[0], parts0[1], DIM // 2)
  x1p = _node0_call(x, agg0, root0, bias0.reshape(1, DIM // 2))

  x1g = [gat[h](src, x1p) for h in (0, 1)]
  msg1 = [_msg1_call(x1g[h], ea, nn1_w1, b1r1, nn1_w2, b2r1, h)
          for h in (0, 1)]
  parts1 = [sc1[h](dst, msg1[h]) for h in (0, 1)]
  agg1 = _reduce_call(parts1[0], parts1[1], DIM)

  out = _readout_call(x1p, agg1, root1p, bias1p, batch_row,
                      fc0_w, fc0_b.reshape(1, -1),
                      fc1_w, fc1_b.reshape(1, -1),
                      fc2_w, fc2_b.reshape(1, -1))
  return out.reshape(-1)


# speedup vs baseline: 2.5539x; 1.0054x over previous
import functools

import jax
import jax.numpy as jnp
from jax import lax
from jax.experimental import pallas as pl
from jax.experimental.pallas import tpu as pltpu
from jax.experimental.pallas import tpu_sc as plsc

N_NODES = 10000
N_EDGES = 160000
F_IN = 128
B_IN = 16
DIM = 16
N_GRAPHS = 64
LANE = 128

NUM_CORES = 2
NUM_SUBCORES = 16
NW = NUM_CORES * NUM_SUBCORES
CHUNK = 128
CHUNKS_PER_W = 40
E_PER_W = CHUNK * CHUNKS_PER_W
E_PAD = NW * E_PER_W
N_ACC = 10240

BLK = 1024
N_EDGE_BLKS = E_PAD // BLK

SPLIT = 4
NCH = CHUNKS_PER_W // SPLIT
E_PER_W_H = NCH * CHUNK
E_HALF = E_PAD // SPLIT
N_EDGE_BLKS_H = E_HALF // BLK
IDX_ROWS_H = E_HALF // CHUNK


def _make_sc_gather(out_w, from_spmem, half):
  mesh = plsc.VectorSubcoreMesh(core_axis_name="c", subcore_axis_name="s")
  rps = 640
  tail = N_NODES - rps * (NUM_SUBCORES - 1)

  nb = 2 if from_spmem else 4
  scratch = [
      pltpu.VMEM((24, CHUNK), jnp.int32),
      pltpu.VMEM((nb, CHUNK, LANE), jnp.float32),
      pltpu.SemaphoreType.DMA,
      pltpu.SemaphoreType.DMA,
      pltpu.SemaphoreType.DMA,
      pltpu.SemaphoreType.DMA,
      pltpu.SemaphoreType.DMA,
      pltpu.SemaphoreType.DMA,
      pltpu.SemaphoreType.DMA,
      pltpu.SemaphoreType.DMA,
  ]
  if from_spmem:
    scratch.append(pltpu.VMEM_SHARED((N_NODES, LANE), jnp.float32))

  @functools.partial(
      pl.kernel,
      mesh=mesh,
      out_type=jax.ShapeDtypeStruct((E_HALF, out_w), jnp.float32),
      scratch_types=scratch,
  )
  def gather_kernel(idx_hbm, table_hbm, out_hbm, idx_v, buf, g0, g1, g2, g3,
                    w0, w1, w2, w3, *maybe_shared):
    c = lax.axis_index("c")
    s = lax.axis_index("s")
    wid = s * NUM_CORES + c
    off = lax.rem(wid * NCH, 8)
    row0 = pl.multiple_of(half * IDX_ROWS_H + wid * NCH - off, 8)
    ebase = wid * E_PER_W_H
    gsems = (g0, g1, g2, g3)
    wsems = (w0, w1, w2, w3)
    if from_spmem:
      table = maybe_shared[0]

      @pl.when(s < NUM_SUBCORES - 1)
      def _():
        pltpu.sync_copy(table_hbm.at[pl.ds(s * rps, rps)],
                        table.at[pl.ds(s * rps, rps)])

      @pl.when(s == NUM_SUBCORES - 1)
      def _():
        pltpu.sync_copy(table_hbm.at[pl.ds((NUM_SUBCORES - 1) * rps, tail)],
                        table.at[pl.ds((NUM_SUBCORES - 1) * rps, tail)])

      pltpu.sync_copy(idx_hbm.at[pl.ds(row0, 24)], idx_v)
      plsc.subcore_barrier()
    else:
      table = table_hbm
      pltpu.sync_copy(idx_hbm.at[pl.ds(row0, 24)], idx_v)
    def src_buf(b):
      if out_w == LANE:
        return buf.at[b]
      return buf.at[b, :, pl.ds(0, out_w)]

    gd, wd = {}, {}
    for j in range(NCH):
      b = j % nb
      if j >= nb:
        wd[j - nb].wait()
      gd[j] = pltpu.async_copy(table.at[idx_v.at[off + j]], buf.at[b],
                               gsems[b])
      if j >= 1:
        gd[j - 1].wait()
        wd[j - 1] = pltpu.async_copy(
            src_buf((j - 1) % nb),
            out_hbm.at[pl.ds(ebase + (j - 1) * CHUNK, CHUNK)],
            wsems[(j - 1) % nb])
    last = NCH - 1
    gd[last].wait()
    wd[last] = pltpu.async_copy(
        src_buf(last % nb), out_hbm.at[pl.ds(ebase + last * CHUNK, CHUNK)],
        wsems[last % nb])
    for j in range(NCH - nb, NCH):
      wd[j].wait()

  return gather_kernel


HALF = N_ACC // 2
ACC_W = HALF * DIM
RED = ACC_W // NUM_SUBCORES


def _make_sc_scatter_add(width, half):
  mesh = plsc.VectorSubcoreMesh(core_axis_name="c", subcore_axis_name="s")
  n_pass = 1 if width == 8 else 2
  rows = N_ACC if width == 8 else HALF

  @functools.partial(
      pl.kernel,
      mesh=mesh,
      out_type=jax.ShapeDtypeStruct((NW, n_pass * ACC_W), jnp.float32),
      compiler_params=pltpu.CompilerParams(needs_layout_passes=False),
      scratch_types=[
          pltpu.VMEM((24, CHUNK), jnp.int32),
          pltpu.VMEM((CHUNK, DIM), jnp.float32),
          pltpu.VMEM((CHUNK, DIM), jnp.float32),
          pltpu.VMEM((ACC_W,), jnp.float32),
          pltpu.SemaphoreType.DMA,
          pltpu.SemaphoreType.DMA,
      ],
  )
  def scatter_kernel(idx_hbm, msg_hbm, out_hbm, idx_v, m_a, m_b, acc,
                     sem_a, sem_b):
    c = lax.axis_index("c")
    s = lax.axis_index("s")
    wid = s * NUM_CORES + c
    off = lax.rem(wid * NCH, 8)
    ebase = wid * E_PER_W_H
    pltpu.sync_copy(
        idx_hbm.at[pl.ds(
            pl.multiple_of(half * IDX_ROWS_H + wid * NCH - off, 8), 24)],
        idx_v)
    lanes = lax.iota(jnp.int32, 16)
    zvec = jnp.zeros((16,), jnp.float32)
    const_mask = lanes < width

    def process_chunk(j, mb, lo):
      @pl.loop(0, CHUNK // 16)
      def _grp(g):
        dvec = idx_v[off + j, pl.ds(g * 16, 16)]
        for l in range(16):
          dd = dvec[l] - lo
          ddv = jnp.full((16,), dd, jnp.int32)
          if n_pass == 1:
            mask = const_mask
          else:
            mask = (ddv >= 0) & (ddv < rows)
          plsc.addupdate_scatter(acc, [ddv * width + lanes],
                                 mb[g * 16 + l, :], mask=mask)

    def start_load(j, buf, sem):
      return pltpu.async_copy(msg_hbm.at[pl.ds(ebase + j * CHUNK, CHUNK)],
                              buf, sem)

    for p in range(n_pass):
      lo = p * rows

      @pl.loop(0, ACC_W // 16, unroll=8)
      def _zero(k):
        acc[pl.ds(k * 16, 16)] = zvec

      start_load(0, m_a, sem_a)

      @pl.loop(0, NCH // 2)
      def _pair(t):
        j = t * 2
        start_load(j + 1, m_b, sem_b)
        pltpu.make_async_copy(msg_hbm.at[pl.ds(ebase, CHUNK)], m_a,
                              sem_a).wait()
        process_chunk(j, m_a, lo)

        @pl.when(t + 1 < NCH // 2)
        def _():
          start_load(j + 2, m_a, sem_a)

        pltpu.make_async_copy(msg_hbm.at[pl.ds(ebase, CHUNK)], m_b,
                              sem_b).wait()
        process_chunk(j + 1, m_b, lo)

      pltpu.sync_copy(acc, out_hbm.at[wid, pl.ds(p * ACC_W, ACC_W)])

  return scatter_kernel


def _reduce_body(*refs):
  out_ref = refs[-1]
  acc = jnp.sum(refs[0][...], axis=0, keepdims=True)
  for r in refs[1:-1]:
    acc = acc + jnp.sum(r[...], axis=0, keepdims=True)
  out_ref[...] = acc


def _reduce_call(parts, width):
  blk = 8192
  total = parts[0].shape[1]
  flat = pl.pallas_call(
      _reduce_body,
      grid=(total // blk,),
      in_specs=[pl.BlockSpec((NW, blk), lambda i: (0, i)) for _ in parts],
      out_specs=pl.BlockSpec((1, blk), lambda i: (0, i)),
      out_shape=jax.ShapeDtypeStruct((1, total), jnp.float32),
  )(*parts)
  return flat.reshape(total // width, width)


def _edge_valid(blk, half):
  rows = (lax.broadcasted_iota(jnp.int32, (blk, 1), 0)
          + (pl.program_id(0) + half * N_EDGE_BLKS_H) * blk)
  return (rows < N_EDGES).astype(jnp.float32)


def _make_msg0_body(half):
  def _msg0_body(xg_ref, ea_ref, w1_ref, b1_ref, w2cat_ref, b2m_ref, out_ref):
    xg = xg_ref[...]
    ea = ea_ref[...]
    h = jnp.maximum(
        lax.dot_general(ea, w1_ref[...], (((1,), (1,)), ((), ())))
        + b1_ref[...], 0.0)
    cols = []
    for o in range(DIM // 2):
      p = jnp.dot(xg, w2cat_ref[:, o * 128:(o + 1) * 128],
                  preferred_element_type=jnp.float32)
      cols.append(jnp.sum(p * h, axis=1, keepdims=True))
    zeros = jnp.zeros((xg.shape[0], DIM // 2), jnp.float32)
    msg = jnp.concatenate(cols + [zeros], axis=1) + jnp.dot(
        xg, b2m_ref[...], preferred_element_type=jnp.float32)
    out_ref[...] = msg * _edge_valid(xg.shape[0], half)
  return _msg0_body


def _make_msg1_body(half):
  def _msg1_body(x1g_ref, ea_ref, w1_ref, b1_ref, w2_ref, b2_ref, out_ref):
    ea = ea_ref[...]
    h = jnp.maximum(
        lax.dot_general(ea, w1_ref[...], (((1,), (1,)), ((), ())))
        + b1_ref[...], 0.0)
    w1e = lax.dot_general(h, w2_ref[...], (((1,), (1,)), ((), ())))
    msg = jnp.zeros((ea.shape[0], DIM), jnp.float32)
    for i in range(DIM // 2):
      msg = msg + x1g_ref[:, i:i + 1] * (
          w1e[:, i * DIM:(i + 1) * DIM] + b2_ref[0:1, i * DIM:(i + 1) * DIM])
    out_ref[...] = msg * _edge_valid(ea.shape[0], half)
  return _msg1_body


def _node0_body(x_ref, aggp_ref, root_ref, bias_ref, out_ref):
  agg = aggp_ref[:N_NODES, :]
  x1 = jnp.maximum(
      jnp.dot(x_ref[...], root_ref[...], preferred_element_type=jnp.float32)
      + agg + bias_ref[...], 0.0)
  out_ref[...] = jnp.concatenate(
      [x1, jnp.zeros((N_NODES, LANE - DIM // 2), jnp.float32)], axis=1)


def _readout_body(x1p_ref, aggp_ref, root_ref, bias_ref, batch_ref,
                  fc0w_ref, fc0b_ref, fc1w_ref, fc1b_ref, fc2w_ref, fc2b_ref,
                  out_ref):
  agg = aggp_ref[:N_NODES, :]
  x2 = jnp.maximum(
      jnp.dot(x1p_ref[...], root_ref[...], preferred_element_type=jnp.float32)
      + agg + bias_ref[...], 0.0)
  gids = lax.broadcasted_iota(jnp.int32, (N_GRAPHS, N_NODES), 0)
  onehot = (gids == batch_ref[...]).astype(jnp.float32)
  x2a = jnp.concatenate([x2, jnp.ones((N_NODES, 1), jnp.float32)], axis=1)
  seg = jnp.dot(onehot, x2a, preferred_element_type=jnp.float32)
  cnt = seg[:, DIM:DIM + 1]
  g = seg[:, :DIM] / jnp.maximum(cnt, 1.0)
  g = jnp.maximum(
      lax.dot_general(g, fc0w_ref[...], (((1,), (1,)), ((), ())))
      + fc0b_ref[...], 0.0)
  g = jnp.maximum(
      lax.dot_general(g, fc1w_ref[...], (((1,), (1,)), ((), ())))
      + fc1b_ref[...], 0.0)
  out_ref[...] = jnp.sum(g * fc2w_ref[...], axis=1, keepdims=True) + fc2b_ref[0, 0]


def _full_spec(shape):
  nd = len(shape)
  return pl.BlockSpec(shape, lambda i=0, *, _n=nd: (0,) * _n)


def _msg0_call(xg, ea, w1, b1, w2cat, b2m, half):
  in_specs = [pl.BlockSpec((BLK, LANE), lambda i: (i, 0)),
              pl.BlockSpec((BLK, B_IN),
                           lambda i, _h=half: (i + _h * N_EDGE_BLKS_H, 0)),
              _full_spec(w1.shape), _full_spec(b1.shape),
              _full_spec(w2cat.shape), _full_spec(b2m.shape)]
  return pl.pallas_call(
      _make_msg0_body(half),
      grid=(N_EDGE_BLKS_H,),
      in_specs=in_specs,
      out_specs=pl.BlockSpec((BLK, DIM), lambda i: (i, 0)),
      out_shape=jax.ShapeDtypeStruct((E_HALF, DIM), jnp.float32),
  )(xg, ea, w1, b1, w2cat, b2m)


def _msg1_call(x1g, ea, w1, b1, w2, b2, half):
  in_specs = [pl.BlockSpec((BLK, LANE), lambda i: (i, 0)),
              pl.BlockSpec((BLK, B_IN),
                           lambda i, _h=half: (i + _h * N_EDGE_BLKS_H, 0)),
              _full_spec(w1.shape), _full_spec(b1.shape),
              _full_spec(w2.shape), _full_spec(b2.shape)]
  return pl.pallas_call(
      _make_msg1_body(half),
      grid=(N_EDGE_BLKS_H,),
      in_specs=in_specs,
      out_specs=pl.BlockSpec((BLK, DIM), lambda i: (i, 0)),
      out_shape=jax.ShapeDtypeStruct((E_HALF, DIM), jnp.float32),
  )(x1g, ea, w1, b1, w2, b2)


def _node0_call(x, aggp, rootp, biasp):
  return pl.pallas_call(
      _node0_body,
      in_specs=[_full_spec(x.shape), _full_spec(aggp.shape),
                _full_spec(rootp.shape), _full_spec(biasp.shape)],
      out_specs=_full_spec((N_NODES, LANE)),
      out_shape=jax.ShapeDtypeStruct((N_NODES, LANE), jnp.float32),
  )(x, aggp, rootp, biasp)


def _readout_call(x1p, aggp, rootp, biasp, batch_row, fc0w, fc0b, fc1w, fc1b,
                  fc2w, fc2b):
  args = (x1p, aggp, rootp, biasp, batch_row, fc0w, fc0b, fc1w, fc1b, fc2w,
          fc2b)
  return pl.pallas_call(
      _readout_body,
      in_specs=[_full_spec(a.shape) for a in args],
      out_specs=_full_spec((N_GRAPHS, 1)),
      out_shape=jax.ShapeDtypeStruct((N_GRAPHS, 1), jnp.float32),
  )(*args)


_make_sc_gather = functools.lru_cache(maxsize=None)(_make_sc_gather)
_make_sc_scatter_add = functools.lru_cache(maxsize=None)(_make_sc_scatter_add)


@jax.jit
def kernel(x, edge_index, edge_attr, batch, nn0_w1, nn0_b1, nn0_w2, nn0_b2,
           root0, bias0, nn1_w1, nn1_b1, nn1_w2, nn1_b2, root1, bias1,
           fc0_w, fc0_b, fc1_w, fc1_b, fc2_w, fc2_b):
  pad_e = E_PAD - N_EDGES
  src = jnp.concatenate(
      [edge_index[0],
       jnp.zeros((pad_e + 8 * CHUNK,), jnp.int32)]).reshape(-1, CHUNK)
  dst = jnp.concatenate(
      [edge_index[1],
       jnp.full((pad_e + 8 * CHUNK,), N_NODES, jnp.int32)]).reshape(-1, CHUNK)
  ea = jnp.concatenate(
      [edge_attr, jnp.zeros((pad_e, B_IN), jnp.float32)], axis=0)
  w2cat0 = nn0_w2.reshape(F_IN, (DIM // 2) * F_IN)
  b2m0 = jnp.pad(nn0_b2.reshape(F_IN, DIM // 2), ((0, 0), (0, DIM // 2)))
  root1p = jnp.pad(root1, ((0, LANE - DIM // 2), (0, 0)))
  bias1p = bias1.reshape(1, DIM)
  b1r0 = nn0_b1.reshape(1, 128)
  b1r1 = nn1_b1.reshape(1, 128)
  b2r1 = nn1_b2.reshape(1, 128)
  batch_row = batch.reshape(1, N_NODES)

  gat = [_make_sc_gather(LANE, True, h) for h in range(SPLIT)]
  sc0 = [_make_sc_scatter_add(8, h) for h in range(SPLIT)]
  sc1 = [_make_sc_scatter_add(16, h) for h in range(SPLIT)]

  xg = [gat[h](src, x) for h in range(SPLIT)]
  msg0 = [_msg0_call(xg[h], ea, nn0_w1, b1r0, w2cat0, b2m0, h)
          for h in range(SPLIT)]
  parts0 = [sc0[h](dst, msg0[h]) for h in range(SPLIT)]
  agg0 = _reduce_call(parts0, DIM // 2)
  x1p = _node0_call(x, agg0, root0, bias0.reshape(1, DIM // 2))

  x1g = [gat[h](src, x1p) for h in range(SPLIT)]
  msg1 = [_msg1_call(x1g[h], ea, nn1_w1, b1r1, nn1_w2, b2r1, h)
          for h in range(SPLIT)]
  parts1 = [sc1[h](dst, msg1[h]) for h in range(SPLIT)]
  agg1 = _reduce_call(parts1, DIM)

  out = _readout_call(x1p, agg1, root1p, bias1p, batch_row,
                      fc0_w, fc0_b.reshape(1, -1),
                      fc1_w, fc1_b.reshape(1, -1),
                      fc2_w, fc2_b.reshape(1, -1))
  return out.reshape(-1)
